# Initial kernel scaffold; baseline (speedup 1.0000x reference)
import jax
import jax.numpy as jnp
from jax.experimental import pallas as pl


def kernel(source_edge_index, target_edge_index, link, user_emb, src_item_emb, tgt_item_emb, src_conv_W, src_conv_b, tgt_conv_W, tgt_conv_b, mix_W, mix_b, pred_W, pred_b):
    raise NotImplementedError("write your pallas kernel here")


# speedup vs baseline: 8.4545x; 8.4545x over previous
import functools

import jax
import jax.numpy as jnp
from jax import lax
from jax.experimental import pallas as pl
from jax.experimental.pallas import tpu as pltpu
from jax.experimental.pallas import tpu_sc as plsc

N_USERS = 20000
N_NODES = 50000
D = 64
HD = 32
E_EDGES = 800000
B_LINKS = 16384

NC = 2
NS = 16
CHUNK = 128
NSUB = E_EDGES // CHUNK
SUB_PER_TILE = -(-NSUB // NS)
ROWS_PER_TILE = N_NODES // NS
ZROWS = 3125

_mesh = plsc.VectorSubcoreMesh(core_axis_name="c", subcore_axis_name="s")
_sc_params = pltpu.CompilerParams(use_tc_tiling_on_sc=False,
                                  needs_layout_passes=False)


GRP = 16
GROUPS_PER_TILE = -(-SUB_PER_TILE // GRP)
GPAIRS = -(-GROUPS_PER_TILE // 2)


def _edge_loop(pk2d, x_hbm, acc, ibufs, isems, rbufs, rsems, tid):
    my0 = tid * SUB_PER_TILE
    myend = jnp.minimum(my0 + SUB_PER_TILE, NSUB)

    def is_full(c0):
        return (c0 + GRP) <= myend

    def stage(c0, p):
        return pltpu.async_copy(pk2d.at[pl.ds(c0, GRP)], ibufs[p], isems[p])

    def gather(ib, b):
        return pltpu.async_copy(x_hbm.at[ib.at[b, 0]], rbufs[b % 2], rsems[b % 2])

    def full_group(g, p):
        c0 = my0 + g * GRP
        ib = ibufs[p]
        pltpu.make_async_copy(pk2d.at[pl.ds(c0, GRP)], ibufs[p], isems[p]).wait()

        @pl.when(is_full(my0 + (g + 1) * GRP))
        def _():
            stage(my0 + (g + 1) * GRP, 1 - p)

        h = gather(ib, 0)
        for b in range(GRP):
            if b + 1 < GRP:
                h_next = gather(ib, b + 1)
            h.wait()
            pltpu.sync_copy(rbufs[b % 2], acc.at[ib.at[b, 1]], add=True)
            if b + 1 < GRP:
                h = h_next

    def tail_group(g, p):
        c0 = my0 + g * GRP
        ib = ibufs[p]

        def tail(b, carry2):
            sub = c0 + b

            @pl.when(sub < myend)
            def _():
                pltpu.sync_copy(pk2d.at[pl.ds(sub, 1)], ib.at[pl.ds(0, 1)])
                pltpu.async_copy(x_hbm.at[ib.at[0, 0]], rbufs[0], rsems[0]).wait()
                pltpu.sync_copy(rbufs[0], acc.at[ib.at[0, 1]], add=True)

            return carry2

        lax.fori_loop(0, GRP, tail, 0)

    def body(k, carry):
        for p in (0, 1):
            g = 2 * k + p
            if 2 * GPAIRS > GROUPS_PER_TILE and p == 1:
                guard_g = g < GROUPS_PER_TILE
            else:
                guard_g = None
            c0 = my0 + g * GRP
            cond_full = is_full(c0)
            if guard_g is not None:
                cond_full = jnp.logical_and(guard_g, cond_full)
                cond_tail = jnp.logical_and(guard_g, jnp.logical_not(is_full(c0)))
            else:
                cond_tail = jnp.logical_not(is_full(c0))

            @pl.when(cond_full)
            def _(g=g, p=p):
                full_group(g, p)

            @pl.when(cond_tail)
            def _(g=g, p=p):
                tail_group(g, p)

        return carry

    @pl.when(is_full(my0))
    def _():
        stage(my0, 0)

    lax.fori_loop(0, GPAIRS, body, 0)


def _one_graph(pk2d, xlo, xhi, outlo, outhi, zc, acc, ibufs, isems, rbufs,
               rsems, c, s):
    pltpu.sync_copy(zc, acc.at[pl.ds(s * ROWS_PER_TILE, ZROWS)])
    plsc.subcore_barrier()

    @pl.when(c == 0)
    def _():
        _edge_loop(pk2d, xlo, acc, ibufs, isems, rbufs, rsems, s)

    @pl.when(c == 1)
    def _():
        _edge_loop(pk2d, xhi, acc, ibufs, isems, rbufs, rsems, s)

    plsc.subcore_barrier()
    r0 = s * ROWS_PER_TILE

    @pl.when(c == 0)
    def _():
        pltpu.sync_copy(acc.at[pl.ds(r0, ROWS_PER_TILE)], outlo.at[pl.ds(r0, ROWS_PER_TILE)])

    @pl.when(c == 1)
    def _():
        pltpu.sync_copy(acc.at[pl.ds(r0, ROWS_PER_TILE)], outhi.at[pl.ds(r0, ROWS_PER_TILE)])


@functools.partial(
    pl.kernel,
    mesh=_mesh,
    compiler_params=_sc_params,
    out_type=[
        jax.ShapeDtypeStruct((N_NODES, HD), jnp.float32),
        jax.ShapeDtypeStruct((N_NODES, HD), jnp.float32),
        jax.ShapeDtypeStruct((N_NODES, HD), jnp.float32),
        jax.ShapeDtypeStruct((N_NODES, HD), jnp.float32),
    ],
    scratch_types=[
        pltpu.VMEM((GRP, 2, CHUNK), jnp.int32),
        pltpu.VMEM((GRP, 2, CHUNK), jnp.int32),
        pltpu.VMEM((CHUNK, HD), jnp.float32),
        pltpu.VMEM((CHUNK, HD), jnp.float32),
        pltpu.VMEM_SHARED((N_NODES, HD), jnp.float32),
        pltpu.SemaphoreType.DMA,
        pltpu.SemaphoreType.DMA,
        pltpu.SemaphoreType.DMA,
        pltpu.SemaphoreType.DMA,
    ],
)
def _conv_sc(pk_s, pk_t, xs_lo, xs_hi, xt_lo, xt_hi, zc,
             as_lo, as_hi, at_lo, at_hi,
             ibuf0, ibuf1, rows_a, rows_b, acc, isem0, isem1, rsem_a, rsem_b):
    c = lax.axis_index("c")
    s = lax.axis_index("s")
    ibufs = (ibuf0, ibuf1)
    isems = (isem0, isem1)
    rbufs = (rows_a, rows_b)
    rsems = (rsem_a, rsem_b)
    _one_graph(pk_s, xs_lo, xs_hi, as_lo, as_hi, zc, acc, ibufs, isems,
               rbufs, rsems, c, s)
    _one_graph(pk_t, xt_lo, xt_hi, at_lo, at_hi, zc, acc, ibufs, isems,
               rbufs, rsems, c, s)


def _deg_loop(dst2d, acc, idg, ones_v):
    s = lax.axis_index("s")
    my0 = s * SUB_PER_TILE
    myend = jnp.minimum(my0 + SUB_PER_TILE, NSUB)

    def body(g, carry):
        c0 = my0 + g * GRP
        full = (c0 + GRP) <= myend

        @pl.when(full)
        def _():
            pltpu.sync_copy(dst2d.at[pl.ds(c0, GRP)], idg)
            for b in range(GRP):
                pltpu.sync_copy(ones_v, acc.at[idg.at[b]], add=True)

        @pl.when(jnp.logical_not(full))
        def _():
            def tail(b, carry2):
                sub = c0 + b

                @pl.when(sub < myend)
                def _():
                    pltpu.sync_copy(dst2d.at[pl.ds(sub, 1)], idg.at[pl.ds(0, 1)])
                    pltpu.sync_copy(ones_v, acc.at[idg.at[0]], add=True)

                return carry2

            lax.fori_loop(0, GRP, tail, 0)

        return carry

    lax.fori_loop(0, GROUPS_PER_TILE, body, 0)


@functools.partial(
    pl.kernel,
    mesh=_mesh,
    compiler_params=_sc_params,
    out_type=[
        jax.ShapeDtypeStruct((N_NODES, 16), jnp.float32),
        jax.ShapeDtypeStruct((N_NODES, 16), jnp.float32),
    ],
    scratch_types=[
        pltpu.VMEM((GRP, CHUNK), jnp.int32),
        pltpu.VMEM((CHUNK, 16), jnp.float32),
        pltpu.VMEM_SHARED((N_NODES, 16), jnp.float32),
    ],
)
def _deg_sc(dst_s, dst_t, zc16, deg_s, deg_t, idg, ones_v, acc):
    c = lax.axis_index("c")
    s = lax.axis_index("s")

    one = jnp.ones((16,), jnp.float32)
    for r in range(CHUNK):
        ones_v[r, 0:16] = one

    pltpu.sync_copy(zc16, acc.at[pl.ds(s * ROWS_PER_TILE, ROWS_PER_TILE)])
    plsc.subcore_barrier()

    @pl.when(c == 0)
    def _():
        _deg_loop(dst_s, acc, idg, ones_v)

    @pl.when(c == 1)
    def _():
        _deg_loop(dst_t, acc, idg, ones_v)

    plsc.subcore_barrier()
    r0 = s * ROWS_PER_TILE

    @pl.when(c == 0)
    def _():
        pltpu.sync_copy(acc.at[pl.ds(r0, ROWS_PER_TILE)], deg_s.at[pl.ds(r0, ROWS_PER_TILE)])

    @pl.when(c == 1)
    def _():
        pltpu.sync_copy(acc.at[pl.ds(r0, ROWS_PER_TILE)], deg_t.at[pl.ds(r0, ROWS_PER_TILE)])


R_BLK = 1000
N_BLOCKS = N_NODES // R_BLK
U_BLOCKS = N_USERS // R_BLK


def _layer_tc_body(aslo, ashi, atlo, athi, degs, degt, ws, bs, wt, bt,
                   mwlo, mwhi, mb, xslo, xshi, xtlo, xthi):
    i = pl.program_id(0)
    a_s = jnp.concatenate([aslo[...], ashi[...]], axis=1)
    a_t = jnp.concatenate([atlo[...], athi[...]], axis=1)
    inv_s = 1.0 / jnp.maximum(degs[...][:, 0:1], 1.0)
    inv_t = 1.0 / jnp.maximum(degt[...][:, 0:1], 1.0)
    c_s = jnp.dot(a_s * inv_s, ws[...], preferred_element_type=jnp.float32) + bs[...]
    c_t = jnp.dot(a_t * inv_t, wt[...], preferred_element_type=jnp.float32) + bt[...]
    u = (jnp.dot(c_s, mwlo[...], preferred_element_type=jnp.float32)
         + jnp.dot(c_t, mwhi[...], preferred_element_type=jnp.float32) + mb[...])
    is_user = (i < U_BLOCKS)
    o_s = jnp.where(is_user, u, c_s)
    o_t = jnp.where(is_user, u, c_t)
    xslo[...] = o_s[:, :HD]
    xshi[...] = o_s[:, HD:]
    xtlo[...] = o_t[:, :HD]
    xthi[...] = o_t[:, HD:]


def _layer_tc(as_lo, as_hi, at_lo, at_hi, deg_s, deg_t, ws, bs, wt, bt, mwlo, mwhi, mb):
    blk = lambda i: (i, 0)
    full = lambda i: (0, 0)
    return pl.pallas_call(
        _layer_tc_body,
        grid=(N_BLOCKS,),
        in_specs=[
            pl.BlockSpec((R_BLK, HD), blk), pl.BlockSpec((R_BLK, HD), blk),
            pl.BlockSpec((R_BLK, HD), blk), pl.BlockSpec((R_BLK, HD), blk),
            pl.BlockSpec((R_BLK, 16), blk), pl.BlockSpec((R_BLK, 16), blk),
            pl.BlockSpec((D, D), full), pl.BlockSpec((1, D), full),
            pl.BlockSpec((D, D), full), pl.BlockSpec((1, D), full),
            pl.BlockSpec((D, D), full), pl.BlockSpec((D, D), full),
            pl.BlockSpec((1, D), full),
        ],
        out_specs=[
            pl.BlockSpec((R_BLK, HD), blk), pl.BlockSpec((R_BLK, HD), blk),
            pl.BlockSpec((R_BLK, HD), blk), pl.BlockSpec((R_BLK, HD), blk),
        ],
        out_shape=[jax.ShapeDtypeStruct((N_NODES, HD), jnp.float32)] * 4,
    )(as_lo, as_hi, at_lo, at_hi, deg_s, deg_t, ws, bs, wt, bt, mwlo, mwhi, mb)


def _mix_pred_tc_body(aslo, ashi, atlo, athi, degs, degt, ws, bs, wt, bt,
                      mwlo, mwhi, mb, ue, u1lo, u1hi, pw, pb, pu, pi):
    a_s = jnp.concatenate([aslo[...], ashi[...]], axis=1)
    a_t = jnp.concatenate([atlo[...], athi[...]], axis=1)
    inv_s = 1.0 / jnp.maximum(degs[...][:, 0:1], 1.0)
    inv_t = 1.0 / jnp.maximum(degt[...][:, 0:1], 1.0)
    c_s = jnp.dot(a_s * inv_s, ws[...], preferred_element_type=jnp.float32) + bs[...]
    c_t = jnp.dot(a_t * inv_t, wt[...], preferred_element_type=jnp.float32) + bt[...]
    u2 = (jnp.dot(c_s, mwlo[...], preferred_element_type=jnp.float32)
          + jnp.dot(c_t, mwhi[...], preferred_element_type=jnp.float32) + mb[...])
    s_blk = jnp.concatenate([ue[...], u1lo[...], u1hi[...], u2], axis=1)
    pwa = pw[...]
    pu[...] = jnp.dot(s_blk, pwa[0:3 * D, :], preferred_element_type=jnp.float32) + pb[...]
    pi[...] = jnp.dot(s_blk, pwa[3 * D:6 * D, :], preferred_element_type=jnp.float32)


def _mix_pred_tc(as_lo, as_hi, at_lo, at_hi, deg_s, deg_t, ws, bs, wt, bt,
                 mwlo, mwhi, mb, ue, u1lo, u1hi, pw, pb):
    blk = lambda i: (i, 0)
    full = lambda i: (0, 0)
    return pl.pallas_call(
        _mix_pred_tc_body,
        grid=(U_BLOCKS,),
        in_specs=[
            pl.BlockSpec((R_BLK, HD), blk), pl.BlockSpec((R_BLK, HD), blk),
            pl.BlockSpec((R_BLK, HD), blk), pl.BlockSpec((R_BLK, HD), blk),
            pl.BlockSpec((R_BLK, 16), blk), pl.BlockSpec((R_BLK, 16), blk),
            pl.BlockSpec((D, D), full), pl.BlockSpec((1, D), full),
            pl.BlockSpec((D, D), full), pl.BlockSpec((1, D), full),
            pl.BlockSpec((D, D), full), pl.BlockSpec((D, D), full),
            pl.BlockSpec((1, D), full),
            pl.BlockSpec((R_BLK, D), blk),
            pl.BlockSpec((R_BLK, HD), blk), pl.BlockSpec((R_BLK, HD), blk),
            pl.BlockSpec((6 * D, 1), full), pl.BlockSpec((1, 1), full),
        ],
        out_specs=[pl.BlockSpec((R_BLK, 1), blk), pl.BlockSpec((R_BLK, 1), blk)],
        out_shape=[jax.ShapeDtypeStruct((N_USERS, 1), jnp.float32)] * 2,
    )(as_lo, as_hi, at_lo, at_hi, deg_s, deg_t, ws, bs, wt, bt, mwlo, mwhi, mb,
      ue, u1lo, u1hi, pw, pb)


LINKS_PER_TILE = B_LINKS // (NC * NS)


@functools.partial(
    pl.kernel,
    mesh=_mesh,
    compiler_params=_sc_params,
    out_type=jax.ShapeDtypeStruct((B_LINKS,), jnp.float32),
    scratch_types=[
        pltpu.VMEM((N_USERS,), jnp.float32),
        pltpu.VMEM((N_USERS,), jnp.float32),
        pltpu.VMEM((LINKS_PER_TILE,), jnp.int32),
        pltpu.VMEM((LINKS_PER_TILE,), jnp.int32),
        pltpu.VMEM((LINKS_PER_TILE,), jnp.float32),
    ],
)
def _head_sc(pu_hbm, pi_hbm, l0_hbm, l1_hbm, out_hbm, pu_v, pi_v, l0_v, l1_v, o_v):
    c = lax.axis_index("c")
    s = lax.axis_index("s")
    wid = s * NC + c
    base = wid * LINKS_PER_TILE
    pltpu.sync_copy(pu_hbm, pu_v)
    pltpu.sync_copy(pi_hbm, pi_v)
    pltpu.sync_copy(l0_hbm.at[pl.ds(base, LINKS_PER_TILE)], l0_v)
    pltpu.sync_copy(l1_hbm.at[pl.ds(base, LINKS_PER_TILE)], l1_v)

    def body(k, carry):
        i0 = l0_v[pl.ds(k * 16, 16)]
        i1 = l1_v[pl.ds(k * 16, 16)]
        g0 = plsc.load_gather(pu_v, [i0])
        g1 = plsc.load_gather(pi_v, [i1])
        z = g0 + g1
        z = jnp.where(z >= 0.0, z, 0.01 * z)
        o_v[pl.ds(k * 16, 16)] = 1.0 / (1.0 + jnp.exp(-z))
        return carry

    lax.fori_loop(0, LINKS_PER_TILE // 16, body, 0)
    pltpu.sync_copy(o_v, out_hbm.at[pl.ds(base, LINKS_PER_TILE)])


def kernel(source_edge_index, target_edge_index, link, user_emb, src_item_emb,
           tgt_item_emb, src_conv_W, src_conv_b, tgt_conv_W, tgt_conv_b,
           mix_W, mix_b, pred_W, pred_b):
    f32 = jnp.float32
    s_src = source_edge_index[0].reshape(NSUB, CHUNK)
    d_src = source_edge_index[1].reshape(NSUB, CHUNK)
    s_tgt = target_edge_index[0].reshape(NSUB, CHUNK)
    d_tgt = target_edge_index[1].reshape(NSUB, CHUNK)
    pk_s = jnp.stack([s_src, d_src], axis=1)
    pk_t = jnp.stack([s_tgt, d_tgt], axis=1)
    l0 = link[0]
    l1 = link[1]

    xs_lo = jnp.concatenate([user_emb[:, :HD], src_item_emb[:, :HD]], axis=0)
    xs_hi = jnp.concatenate([user_emb[:, HD:], src_item_emb[:, HD:]], axis=0)
    xt_lo = jnp.concatenate([user_emb[:, :HD], tgt_item_emb[:, :HD]], axis=0)
    xt_hi = jnp.concatenate([user_emb[:, HD:], tgt_item_emb[:, HD:]], axis=0)

    zc = jnp.zeros((ZROWS, HD), f32)
    zc16 = jnp.zeros((ROWS_PER_TILE, 16), f32)

    deg_s, deg_t = _deg_sc(d_src, d_tgt, zc16)

    ws0, ws1 = src_conv_W[0], src_conv_W[1]
    wt0, wt1 = tgt_conv_W[0], tgt_conv_W[1]
    bs0 = src_conv_b[0].reshape(1, D)
    bs1 = src_conv_b[1].reshape(1, D)
    bt0 = tgt_conv_b[0].reshape(1, D)
    bt1 = tgt_conv_b[1].reshape(1, D)
    mw0lo, mw0hi = mix_W[0][:D], mix_W[0][D:]
    mw1lo, mw1hi = mix_W[1][:D], mix_W[1][D:]
    mb0 = mix_b[0].reshape(1, D)
    mb1 = mix_b[1].reshape(1, D)

    as_lo, as_hi, at_lo, at_hi = _conv_sc(pk_s, pk_t, xs_lo, xs_hi, xt_lo, xt_hi, zc)
    x1s_lo, x1s_hi, x1t_lo, x1t_hi = _layer_tc(
        as_lo, as_hi, at_lo, at_hi, deg_s, deg_t,
        ws0, bs0, wt0, bt0, mw0lo, mw0hi, mb0)

    a2s_lo, a2s_hi, a2t_lo, a2t_hi = _conv_sc(
        pk_s, pk_t, x1s_lo, x1s_hi, x1t_lo, x1t_hi, zc)
    pu, pi = _mix_pred_tc(a2s_lo, a2s_hi, a2t_lo, a2t_hi, deg_s, deg_t,
                          ws1, bs1, wt1, bt1, mw1lo, mw1hi, mb1,
                          user_emb, x1s_lo, x1s_hi, pred_W, pred_b.reshape(1, 1))

    out = _head_sc(pu.reshape(N_USERS), pi.reshape(N_USERS), l0, l1)
    return out.reshape(B_LINKS, 1)


# speedup vs baseline: 8.7198x; 1.0314x over previous
import functools

import jax
import jax.numpy as jnp
from jax import lax
from jax.experimental import pallas as pl
from jax.experimental.pallas import tpu as pltpu
from jax.experimental.pallas import tpu_sc as plsc

N_USERS = 20000
N_NODES = 50000
D = 64
HD = 32
E_EDGES = 800000
B_LINKS = 16384

NC = 2
NS = 16
CHUNK = 128
NSUB = E_EDGES // CHUNK
SUB_PER_TILE = -(-NSUB // NS)
ROWS_PER_TILE = N_NODES // NS
ZROWS = 3125

_mesh = plsc.VectorSubcoreMesh(core_axis_name="c", subcore_axis_name="s")
_sc_params = pltpu.CompilerParams(use_tc_tiling_on_sc=False,
                                  needs_layout_passes=False)


GRP = 16
GROUPS_PER_TILE = -(-SUB_PER_TILE // GRP)
GPAIRS = -(-GROUPS_PER_TILE // 2)


def _edge_loop(pk2d, x_hbm, acc, ibufs, isems, rbufs, rsems, tid):
    my0 = tid * SUB_PER_TILE
    myend = jnp.minimum(my0 + SUB_PER_TILE, NSUB)

    def is_full(c0):
        return (c0 + GRP) <= myend

    def stage(c0, p):
        return pltpu.async_copy(pk2d.at[pl.ds(c0, GRP)], ibufs[p], isems[p])

    def gather(ib, b):
        return pltpu.async_copy(x_hbm.at[ib.at[b, 0]], rbufs[b % 2], rsems[b % 2])

    def full_group(g, p):
        c0 = my0 + g * GRP
        ib = ibufs[p]
        pltpu.make_async_copy(pk2d.at[pl.ds(c0, GRP)], ibufs[p], isems[p]).wait()

        @pl.when(is_full(my0 + (g + 1) * GRP))
        def _():
            stage(my0 + (g + 1) * GRP, 1 - p)

        h = gather(ib, 0)
        for b in range(GRP):
            if b + 1 < GRP:
                h_next = gather(ib, b + 1)
            h.wait()
            pltpu.sync_copy(rbufs[b % 2], acc.at[ib.at[b, 1]], add=True)
            if b + 1 < GRP:
                h = h_next

    def tail_group(g, p):
        c0 = my0 + g * GRP
        ib = ibufs[p]

        def tail(b, carry2):
            sub = c0 + b

            @pl.when(sub < myend)
            def _():
                pltpu.sync_copy(pk2d.at[pl.ds(sub, 1)], ib.at[pl.ds(0, 1)])
                pltpu.async_copy(x_hbm.at[ib.at[0, 0]], rbufs[0], rsems[0]).wait()
                pltpu.sync_copy(rbufs[0], acc.at[ib.at[0, 1]], add=True)

            return carry2

        lax.fori_loop(0, GRP, tail, 0)

    def body(k, carry):
        for p in (0, 1):
            g = 2 * k + p
            if 2 * GPAIRS > GROUPS_PER_TILE and p == 1:
                guard_g = g < GROUPS_PER_TILE
            else:
                guard_g = None
            c0 = my0 + g * GRP
            cond_full = is_full(c0)
            if guard_g is not None:
                cond_full = jnp.logical_and(guard_g, cond_full)
                cond_tail = jnp.logical_and(guard_g, jnp.logical_not(is_full(c0)))
            else:
                cond_tail = jnp.logical_not(is_full(c0))

            @pl.when(cond_full)
            def _(g=g, p=p):
                full_group(g, p)

            @pl.when(cond_tail)
            def _(g=g, p=p):
                tail_group(g, p)

        return carry

    @pl.when(is_full(my0))
    def _():
        stage(my0, 0)

    lax.fori_loop(0, GPAIRS, body, 0)


def _one_graph(pk2d, xlo, xhi, outlo, outhi, zc, acc, ibufs, isems, rbufs,
               rsems, c, s):
    pltpu.sync_copy(zc, acc.at[pl.ds(s * ROWS_PER_TILE, ZROWS)])
    plsc.subcore_barrier()

    @pl.when(c == 0)
    def _():
        _edge_loop(pk2d, xlo, acc, ibufs, isems, rbufs, rsems, s)

    @pl.when(c == 1)
    def _():
        _edge_loop(pk2d, xhi, acc, ibufs, isems, rbufs, rsems, s)

    plsc.subcore_barrier()
    r0 = s * ROWS_PER_TILE

    @pl.when(c == 0)
    def _():
        pltpu.sync_copy(acc.at[pl.ds(r0, ROWS_PER_TILE)], outlo.at[pl.ds(r0, ROWS_PER_TILE)])

    @pl.when(c == 1)
    def _():
        pltpu.sync_copy(acc.at[pl.ds(r0, ROWS_PER_TILE)], outhi.at[pl.ds(r0, ROWS_PER_TILE)])


@functools.partial(
    pl.kernel,
    mesh=_mesh,
    compiler_params=_sc_params,
    out_type=[
        jax.ShapeDtypeStruct((N_NODES, HD), jnp.float32),
        jax.ShapeDtypeStruct((N_NODES, HD), jnp.float32),
        jax.ShapeDtypeStruct((N_NODES, HD), jnp.float32),
        jax.ShapeDtypeStruct((N_NODES, HD), jnp.float32),
    ],
    scratch_types=[
        pltpu.VMEM((GRP, 2, CHUNK), jnp.int32),
        pltpu.VMEM((GRP, 2, CHUNK), jnp.int32),
        pltpu.VMEM((CHUNK, HD), jnp.float32),
        pltpu.VMEM((CHUNK, HD), jnp.float32),
        pltpu.VMEM_SHARED((N_NODES, HD), jnp.float32),
        pltpu.SemaphoreType.DMA,
        pltpu.SemaphoreType.DMA,
        pltpu.SemaphoreType.DMA,
        pltpu.SemaphoreType.DMA,
    ],
)
def _conv_sc(pk_s, pk_t, xs_lo, xs_hi, xt_lo, xt_hi, zc, ord_dep,
             as_lo, as_hi, at_lo, at_hi,
             ibuf0, ibuf1, rows_a, rows_b, acc, isem0, isem1, rsem_a, rsem_b):
    c = lax.axis_index("c")
    s = lax.axis_index("s")
    ibufs = (ibuf0, ibuf1)
    isems = (isem0, isem1)
    rbufs = (rows_a, rows_b)
    rsems = (rsem_a, rsem_b)
    _one_graph(pk_s, xs_lo, xs_hi, as_lo, as_hi, zc, acc, ibufs, isems,
               rbufs, rsems, c, s)
    _one_graph(pk_t, xt_lo, xt_hi, at_lo, at_hi, zc, acc, ibufs, isems,
               rbufs, rsems, c, s)


def _deg_loop(dst2d, acc, idg, ones_v):
    s = lax.axis_index("s")
    my0 = s * SUB_PER_TILE
    myend = jnp.minimum(my0 + SUB_PER_TILE, NSUB)

    def body(g, carry):
        c0 = my0 + g * GRP
        full = (c0 + GRP) <= myend

        @pl.when(full)
        def _():
            pltpu.sync_copy(dst2d.at[pl.ds(c0, GRP)], idg)
            for b in range(GRP):
                pltpu.sync_copy(ones_v, acc.at[idg.at[b]], add=True)

        @pl.when(jnp.logical_not(full))
        def _():
            def tail(b, carry2):
                sub = c0 + b

                @pl.when(sub < myend)
                def _():
                    pltpu.sync_copy(dst2d.at[pl.ds(sub, 1)], idg.at[pl.ds(0, 1)])
                    pltpu.sync_copy(ones_v, acc.at[idg.at[0]], add=True)

                return carry2

            lax.fori_loop(0, GRP, tail, 0)

        return carry

    lax.fori_loop(0, GROUPS_PER_TILE, body, 0)


@functools.partial(
    pl.kernel,
    mesh=_mesh,
    compiler_params=_sc_params,
    out_type=[
        jax.ShapeDtypeStruct((N_NODES, 16), jnp.float32),
        jax.ShapeDtypeStruct((N_NODES, 16), jnp.float32),
    ],
    scratch_types=[
        pltpu.VMEM((GRP, CHUNK), jnp.int32),
        pltpu.VMEM((CHUNK, 16), jnp.float32),
        pltpu.VMEM_SHARED((N_NODES, 16), jnp.float32),
    ],
)
def _deg_sc(dst_s, dst_t, zc16, deg_s, deg_t, idg, ones_v, acc):
    c = lax.axis_index("c")
    s = lax.axis_index("s")

    one = jnp.ones((16,), jnp.float32)
    for r in range(CHUNK):
        ones_v[r, 0:16] = one

    pltpu.sync_copy(zc16, acc.at[pl.ds(s * ROWS_PER_TILE, ROWS_PER_TILE)])
    plsc.subcore_barrier()

    @pl.when(c == 0)
    def _():
        _deg_loop(dst_s, acc, idg, ones_v)

    @pl.when(c == 1)
    def _():
        _deg_loop(dst_t, acc, idg, ones_v)

    plsc.subcore_barrier()
    r0 = s * ROWS_PER_TILE

    @pl.when(c == 0)
    def _():
        pltpu.sync_copy(acc.at[pl.ds(r0, ROWS_PER_TILE)], deg_s.at[pl.ds(r0, ROWS_PER_TILE)])

    @pl.when(c == 1)
    def _():
        pltpu.sync_copy(acc.at[pl.ds(r0, ROWS_PER_TILE)], deg_t.at[pl.ds(r0, ROWS_PER_TILE)])


R_BLK = 5000
N_BLOCKS = N_NODES // R_BLK
U_BLOCKS = N_USERS // R_BLK


def _layer_tc_body(aslo, ashi, atlo, athi, degs, degt, ws, bs, wt, bt,
                   mwlo, mwhi, mb, xslo, xshi, xtlo, xthi):
    i = pl.program_id(0)
    a_s = jnp.concatenate([aslo[...], ashi[...]], axis=1)
    a_t = jnp.concatenate([atlo[...], athi[...]], axis=1)
    inv_s = 1.0 / jnp.maximum(degs[...][:, 0:1], 1.0)
    inv_t = 1.0 / jnp.maximum(degt[...][:, 0:1], 1.0)
    c_s = jnp.dot(a_s * inv_s, ws[...], preferred_element_type=jnp.float32) + bs[...]
    c_t = jnp.dot(a_t * inv_t, wt[...], preferred_element_type=jnp.float32) + bt[...]
    u = (jnp.dot(c_s, mwlo[...], preferred_element_type=jnp.float32)
         + jnp.dot(c_t, mwhi[...], preferred_element_type=jnp.float32) + mb[...])
    is_user = (i < U_BLOCKS)
    o_s = jnp.where(is_user, u, c_s)
    o_t = jnp.where(is_user, u, c_t)
    xslo[...] = o_s[:, :HD]
    xshi[...] = o_s[:, HD:]
    xtlo[...] = o_t[:, :HD]
    xthi[...] = o_t[:, HD:]


def _layer_tc(as_lo, as_hi, at_lo, at_hi, deg_s, deg_t, ws, bs, wt, bt, mwlo, mwhi, mb):
    blk = lambda i: (i, 0)
    full = lambda i: (0, 0)
    return pl.pallas_call(
        _layer_tc_body,
        grid=(N_BLOCKS,),
        in_specs=[
            pl.BlockSpec((R_BLK, HD), blk), pl.BlockSpec((R_BLK, HD), blk),
            pl.BlockSpec((R_BLK, HD), blk), pl.BlockSpec((R_BLK, HD), blk),
            pl.BlockSpec((R_BLK, 16), blk), pl.BlockSpec((R_BLK, 16), blk),
            pl.BlockSpec((D, D), full), pl.BlockSpec((1, D), full),
            pl.BlockSpec((D, D), full), pl.BlockSpec((1, D), full),
            pl.BlockSpec((D, D), full), pl.BlockSpec((D, D), full),
            pl.BlockSpec((1, D), full),
        ],
        out_specs=[
            pl.BlockSpec((R_BLK, HD), blk), pl.BlockSpec((R_BLK, HD), blk),
            pl.BlockSpec((R_BLK, HD), blk), pl.BlockSpec((R_BLK, HD), blk),
        ],
        out_shape=[jax.ShapeDtypeStruct((N_NODES, HD), jnp.float32)] * 4,
    )(as_lo, as_hi, at_lo, at_hi, deg_s, deg_t, ws, bs, wt, bt, mwlo, mwhi, mb)


def _mix_pred_tc_body(aslo, ashi, atlo, athi, degs, degt, ws, bs, wt, bt,
                      mwlo, mwhi, mb, ue, u1lo, u1hi, pw, pb, pu, pi):
    a_s = jnp.concatenate([aslo[...], ashi[...]], axis=1)
    a_t = jnp.concatenate([atlo[...], athi[...]], axis=1)
    inv_s = 1.0 / jnp.maximum(degs[...][:, 0:1], 1.0)
    inv_t = 1.0 / jnp.maximum(degt[...][:, 0:1], 1.0)
    c_s = jnp.dot(a_s * inv_s, ws[...], preferred_element_type=jnp.float32) + bs[...]
    c_t = jnp.dot(a_t * inv_t, wt[...], preferred_element_type=jnp.float32) + bt[...]
    u2 = (jnp.dot(c_s, mwlo[...], preferred_element_type=jnp.float32)
          + jnp.dot(c_t, mwhi[...], preferred_element_type=jnp.float32) + mb[...])
    s_blk = jnp.concatenate([ue[...], u1lo[...], u1hi[...], u2], axis=1)
    pwa = pw[...]
    pu[...] = jnp.dot(s_blk, pwa[0:3 * D, :], preferred_element_type=jnp.float32) + pb[...]
    pi[...] = jnp.dot(s_blk, pwa[3 * D:6 * D, :], preferred_element_type=jnp.float32)


def _mix_pred_tc(as_lo, as_hi, at_lo, at_hi, deg_s, deg_t, ws, bs, wt, bt,
                 mwlo, mwhi, mb, ue, u1lo, u1hi, pw, pb):
    blk = lambda i: (i, 0)
    full = lambda i: (0, 0)
    return pl.pallas_call(
        _mix_pred_tc_body,
        grid=(U_BLOCKS,),
        in_specs=[
            pl.BlockSpec((R_BLK, HD), blk), pl.BlockSpec((R_BLK, HD), blk),
            pl.BlockSpec((R_BLK, HD), blk), pl.BlockSpec((R_BLK, HD), blk),
            pl.BlockSpec((R_BLK, 16), blk), pl.BlockSpec((R_BLK, 16), blk),
            pl.BlockSpec((D, D), full), pl.BlockSpec((1, D), full),
            pl.BlockSpec((D, D), full), pl.BlockSpec((1, D), full),
            pl.BlockSpec((D, D), full), pl.BlockSpec((D, D), full),
            pl.BlockSpec((1, D), full),
            pl.BlockSpec((R_BLK, D), blk),
            pl.BlockSpec((R_BLK, HD), blk), pl.BlockSpec((R_BLK, HD), blk),
            pl.BlockSpec((6 * D, 1), full), pl.BlockSpec((1, 1), full),
        ],
        out_specs=[pl.BlockSpec((R_BLK, 1), blk), pl.BlockSpec((R_BLK, 1), blk)],
        out_shape=[jax.ShapeDtypeStruct((N_USERS, 1), jnp.float32)] * 2,
    )(as_lo, as_hi, at_lo, at_hi, deg_s, deg_t, ws, bs, wt, bt, mwlo, mwhi, mb,
      ue, u1lo, u1hi, pw, pb)


LINKS_PER_TILE = B_LINKS // (NC * NS)


@functools.partial(
    pl.kernel,
    mesh=_mesh,
    compiler_params=_sc_params,
    out_type=jax.ShapeDtypeStruct((B_LINKS,), jnp.float32),
    scratch_types=[
        pltpu.VMEM((N_USERS,), jnp.float32),
        pltpu.VMEM((N_USERS,), jnp.float32),
        pltpu.VMEM((LINKS_PER_TILE,), jnp.int32),
        pltpu.VMEM((LINKS_PER_TILE,), jnp.int32),
        pltpu.VMEM((LINKS_PER_TILE,), jnp.float32),
    ],
)
def _head_sc(pu_hbm, pi_hbm, l0_hbm, l1_hbm, out_hbm, pu_v, pi_v, l0_v, l1_v, o_v):
    c = lax.axis_index("c")
    s = lax.axis_index("s")
    wid = s * NC + c
    base = wid * LINKS_PER_TILE
    pltpu.sync_copy(pu_hbm, pu_v)
    pltpu.sync_copy(pi_hbm, pi_v)
    pltpu.sync_copy(l0_hbm.at[pl.ds(base, LINKS_PER_TILE)], l0_v)
    pltpu.sync_copy(l1_hbm.at[pl.ds(base, LINKS_PER_TILE)], l1_v)

    def body(k, carry):
        i0 = l0_v[pl.ds(k * 16, 16)]
        i1 = l1_v[pl.ds(k * 16, 16)]
        g0 = plsc.load_gather(pu_v, [i0])
        g1 = plsc.load_gather(pi_v, [i1])
        z = g0 + g1
        z = jnp.where(z >= 0.0, z, 0.01 * z)
        o_v[pl.ds(k * 16, 16)] = 1.0 / (1.0 + jnp.exp(-z))
        return carry

    lax.fori_loop(0, LINKS_PER_TILE // 16, body, 0)
    pltpu.sync_copy(o_v, out_hbm.at[pl.ds(base, LINKS_PER_TILE)])


def kernel(source_edge_index, target_edge_index, link, user_emb, src_item_emb,
           tgt_item_emb, src_conv_W, src_conv_b, tgt_conv_W, tgt_conv_b,
           mix_W, mix_b, pred_W, pred_b):
    f32 = jnp.float32
    s_src = source_edge_index[0].reshape(NSUB, CHUNK)
    d_src = source_edge_index[1].reshape(NSUB, CHUNK)
    s_tgt = target_edge_index[0].reshape(NSUB, CHUNK)
    d_tgt = target_edge_index[1].reshape(NSUB, CHUNK)
    pk_s = jnp.stack([s_src, d_src], axis=1)
    pk_t = jnp.stack([s_tgt, d_tgt], axis=1)
    l0 = link[0]
    l1 = link[1]

    xs_lo = jnp.concatenate([user_emb[:, :HD], src_item_emb[:, :HD]], axis=0)
    xs_hi = jnp.concatenate([user_emb[:, HD:], src_item_emb[:, HD:]], axis=0)
    xt_lo = jnp.concatenate([user_emb[:, :HD], tgt_item_emb[:, :HD]], axis=0)
    xt_hi = jnp.concatenate([user_emb[:, HD:], tgt_item_emb[:, HD:]], axis=0)

    zc = jnp.zeros((ZROWS, HD), f32)
    zc16 = jnp.zeros((ROWS_PER_TILE, 16), f32)

    deg_s, deg_t = _deg_sc(d_src, d_tgt, zc16)

    ws0, ws1 = src_conv_W[0], src_conv_W[1]
    wt0, wt1 = tgt_conv_W[0], tgt_conv_W[1]
    bs0 = src_conv_b[0].reshape(1, D)
    bs1 = src_conv_b[1].reshape(1, D)
    bt0 = tgt_conv_b[0].reshape(1, D)
    bt1 = tgt_conv_b[1].reshape(1, D)
    mw0lo, mw0hi = mix_W[0][:D], mix_W[0][D:]
    mw1lo, mw1hi = mix_W[1][:D], mix_W[1][D:]
    mb0 = mix_b[0].reshape(1, D)
    mb1 = mix_b[1].reshape(1, D)

    as_lo, as_hi, at_lo, at_hi = _conv_sc(pk_s, pk_t, xs_lo, xs_hi, xt_lo,
                                          xt_hi, zc, deg_s)
    x1s_lo, x1s_hi, x1t_lo, x1t_hi = _layer_tc(
        as_lo, as_hi, at_lo, at_hi, deg_s, deg_t,
        ws0, bs0, wt0, bt0, mw0lo, mw0hi, mb0)

    a2s_lo, a2s_hi, a2t_lo, a2t_hi = _conv_sc(
        pk_s, pk_t, x1s_lo, x1s_hi, x1t_lo, x1t_hi, zc, deg_s)
    pu, pi = _mix_pred_tc(a2s_lo, a2s_hi, a2t_lo, a2t_hi, deg_s, deg_t,
                          ws1, bs1, wt1, bt1, mw1lo, mw1hi, mb1,
                          user_emb, x1s_lo, x1s_hi, pred_W, pred_b.reshape(1, 1))

    out = _head_sc(pu.reshape(N_USERS), pi.reshape(N_USERS), l0, l1)
    return out.reshape(B_LINKS, 1)


# speedup vs baseline: 9.4640x; 1.0853x over previous
import functools

import jax
import jax.numpy as jnp
from jax import lax
from jax.experimental import pallas as pl
from jax.experimental.pallas import tpu as pltpu
from jax.experimental.pallas import tpu_sc as plsc

N_USERS = 20000
N_NODES = 50000
D = 64
HD = 32
E_EDGES = 800000
B_LINKS = 16384

NC = 2
NS = 16
CHUNK = 128
NSUB = E_EDGES // CHUNK
SUB_PER_TILE = -(-NSUB // NS)
ROWS_PER_TILE = N_NODES // NS
ZROWS = 3125

_mesh = plsc.VectorSubcoreMesh(core_axis_name="c", subcore_axis_name="s")
_sc_params = pltpu.CompilerParams(use_tc_tiling_on_sc=False,
                                  needs_layout_passes=False)


GRP = 16
GROUPS_PER_TILE = -(-SUB_PER_TILE // GRP)
GPAIRS = -(-GROUPS_PER_TILE // 2)


def _edge_loop(pk2d, x_hbm, acc, ibufs, isems, rbufs, rsems, tid):
    my0 = tid * SUB_PER_TILE
    myend = jnp.minimum(my0 + SUB_PER_TILE, NSUB)

    def is_full(c0):
        return (c0 + GRP) <= myend

    def stage(c0, p):
        return pltpu.async_copy(pk2d.at[pl.ds(c0, GRP)], ibufs[p], isems[p])

    def gather(ib, b):
        return pltpu.async_copy(x_hbm.at[ib.at[b, 0]], rbufs[b % 2], rsems[b % 2])

    def full_group(g, p):
        c0 = my0 + g * GRP
        ib = ibufs[p]
        pltpu.make_async_copy(pk2d.at[pl.ds(c0, GRP)], ibufs[p], isems[p]).wait()

        @pl.when(is_full(my0 + (g + 1) * GRP))
        def _():
            stage(my0 + (g + 1) * GRP, 1 - p)

        h = gather(ib, 0)
        for b in range(GRP):
            if b + 1 < GRP:
                h_next = gather(ib, b + 1)
            h.wait()
            pltpu.sync_copy(rbufs[b % 2], acc.at[ib.at[b, 1]], add=True)
            if b + 1 < GRP:
                h = h_next

    def tail_group(g, p):
        c0 = my0 + g * GRP
        ib = ibufs[p]

        def tail(b, carry2):
            sub = c0 + b

            @pl.when(sub < myend)
            def _():
                pltpu.sync_copy(pk2d.at[pl.ds(sub, 1)], ib.at[pl.ds(0, 1)])
                pltpu.async_copy(x_hbm.at[ib.at[0, 0]], rbufs[0], rsems[0]).wait()
                pltpu.sync_copy(rbufs[0], acc.at[ib.at[0, 1]], add=True)

            return carry2

        lax.fori_loop(0, GRP, tail, 0)

    def body(k, carry):
        for p in (0, 1):
            g = 2 * k + p
            if 2 * GPAIRS > GROUPS_PER_TILE and p == 1:
                guard_g = g < GROUPS_PER_TILE
            else:
                guard_g = None
            c0 = my0 + g * GRP
            cond_full = is_full(c0)
            if guard_g is not None:
                cond_full = jnp.logical_and(guard_g, cond_full)
                cond_tail = jnp.logical_and(guard_g, jnp.logical_not(is_full(c0)))
            else:
                cond_tail = jnp.logical_not(is_full(c0))

            @pl.when(cond_full)
            def _(g=g, p=p):
                full_group(g, p)

            @pl.when(cond_tail)
            def _(g=g, p=p):
                tail_group(g, p)

        return carry

    @pl.when(is_full(my0))
    def _():
        stage(my0, 0)

    lax.fori_loop(0, GPAIRS, body, 0)


def _one_graph(pk2d, xlo, xhi, outlo, outhi, zc, acc, ibufs, isems, rbufs,
               rsems, c, s):
    pltpu.sync_copy(zc, acc.at[pl.ds(s * ROWS_PER_TILE, ZROWS)])
    plsc.subcore_barrier()

    @pl.when(c == 0)
    def _():
        _edge_loop(pk2d, xlo, acc, ibufs, isems, rbufs, rsems, s)

    @pl.when(c == 1)
    def _():
        _edge_loop(pk2d, xhi, acc, ibufs, isems, rbufs, rsems, s)

    plsc.subcore_barrier()
    r0 = s * ROWS_PER_TILE

    @pl.when(c == 0)
    def _():
        pltpu.sync_copy(acc.at[pl.ds(r0, ROWS_PER_TILE)], outlo.at[pl.ds(r0, ROWS_PER_TILE)])

    @pl.when(c == 1)
    def _():
        pltpu.sync_copy(acc.at[pl.ds(r0, ROWS_PER_TILE)], outhi.at[pl.ds(r0, ROWS_PER_TILE)])


@functools.partial(
    pl.kernel,
    mesh=_mesh,
    compiler_params=_sc_params,
    out_type=[
        jax.ShapeDtypeStruct((N_NODES, HD), jnp.float32),
        jax.ShapeDtypeStruct((N_NODES, HD), jnp.float32),
    ],
    scratch_types=[
        pltpu.VMEM((GRP, 2, CHUNK), jnp.int32),
        pltpu.VMEM((GRP, 2, CHUNK), jnp.int32),
        pltpu.VMEM((CHUNK, HD), jnp.float32),
        pltpu.VMEM((CHUNK, HD), jnp.float32),
        pltpu.VMEM_SHARED((N_NODES, HD), jnp.float32),
        pltpu.SemaphoreType.DMA,
        pltpu.SemaphoreType.DMA,
        pltpu.SemaphoreType.DMA,
        pltpu.SemaphoreType.DMA,
    ],
)
def _conv_sc(pk, x_lo, x_hi, zc, ord_dep,
             a_lo, a_hi,
             ibuf0, ibuf1, rows_a, rows_b, acc, isem0, isem1, rsem_a, rsem_b):
    c = lax.axis_index("c")
    s = lax.axis_index("s")
    ibufs = (ibuf0, ibuf1)
    isems = (isem0, isem1)
    rbufs = (rows_a, rows_b)
    rsems = (rsem_a, rsem_b)
    _one_graph(pk, x_lo, x_hi, a_lo, a_hi, zc, acc, ibufs, isems,
               rbufs, rsems, c, s)


def _deg_loop(dst2d, acc, idg, ones_v):
    s = lax.axis_index("s")
    my0 = s * SUB_PER_TILE
    myend = jnp.minimum(my0 + SUB_PER_TILE, NSUB)

    def body(g, carry):
        c0 = my0 + g * GRP
        full = (c0 + GRP) <= myend

        @pl.when(full)
        def _():
            pltpu.sync_copy(dst2d.at[pl.ds(c0, GRP)], idg)
            for b in range(GRP):
                pltpu.sync_copy(ones_v, acc.at[idg.at[b]], add=True)

        @pl.when(jnp.logical_not(full))
        def _():
            def tail(b, carry2):
                sub = c0 + b

                @pl.when(sub < myend)
                def _():
                    pltpu.sync_copy(dst2d.at[pl.ds(sub, 1)], idg.at[pl.ds(0, 1)])
                    pltpu.sync_copy(ones_v, acc.at[idg.at[0]], add=True)

                return carry2

            lax.fori_loop(0, GRP, tail, 0)

        return carry

    lax.fori_loop(0, GROUPS_PER_TILE, body, 0)


@functools.partial(
    pl.kernel,
    mesh=_mesh,
    compiler_params=_sc_params,
    out_type=[
        jax.ShapeDtypeStruct((N_NODES, 16), jnp.float32),
        jax.ShapeDtypeStruct((N_NODES, 16), jnp.float32),
    ],
    scratch_types=[
        pltpu.VMEM((GRP, CHUNK), jnp.int32),
        pltpu.VMEM((CHUNK, 16), jnp.float32),
        pltpu.VMEM_SHARED((N_NODES, 16), jnp.float32),
    ],
)
def _deg_sc(dst_s, dst_t, zc16, deg_s, deg_t, idg, ones_v, acc):
    c = lax.axis_index("c")
    s = lax.axis_index("s")

    one = jnp.ones((16,), jnp.float32)
    for r in range(CHUNK):
        ones_v[r, 0:16] = one

    pltpu.sync_copy(zc16, acc.at[pl.ds(s * ROWS_PER_TILE, ROWS_PER_TILE)])
    plsc.subcore_barrier()

    @pl.when(c == 0)
    def _():
        _deg_loop(dst_s, acc, idg, ones_v)

    @pl.when(c == 1)
    def _():
        _deg_loop(dst_t, acc, idg, ones_v)

    plsc.subcore_barrier()
    r0 = s * ROWS_PER_TILE

    @pl.when(c == 0)
    def _():
        pltpu.sync_copy(acc.at[pl.ds(r0, ROWS_PER_TILE)], deg_s.at[pl.ds(r0, ROWS_PER_TILE)])

    @pl.when(c == 1)
    def _():
        pltpu.sync_copy(acc.at[pl.ds(r0, ROWS_PER_TILE)], deg_t.at[pl.ds(r0, ROWS_PER_TILE)])


R_BLK = 5000
N_BLOCKS = N_NODES // R_BLK
U_BLOCKS = N_USERS // R_BLK


def _layer_tc_body(aslo, ashi, atlo, athi, degs, degt, ws, bs, wt, bt,
                   mwlo, mwhi, mb, xslo, xshi, xtlo, xthi):
    i = pl.program_id(0)
    a_s = jnp.concatenate([aslo[...], ashi[...]], axis=1)
    a_t = jnp.concatenate([atlo[...], athi[...]], axis=1)
    inv_s = 1.0 / jnp.maximum(degs[...][:, 0:1], 1.0)
    inv_t = 1.0 / jnp.maximum(degt[...][:, 0:1], 1.0)
    c_s = jnp.dot(a_s * inv_s, ws[...], preferred_element_type=jnp.float32) + bs[...]
    c_t = jnp.dot(a_t * inv_t, wt[...], preferred_element_type=jnp.float32) + bt[...]
    u = (jnp.dot(c_s, mwlo[...], preferred_element_type=jnp.float32)
         + jnp.dot(c_t, mwhi[...], preferred_element_type=jnp.float32) + mb[...])
    is_user = (i < U_BLOCKS)
    o_s = jnp.where(is_user, u, c_s)
    o_t = jnp.where(is_user, u, c_t)
    xslo[...] = o_s[:, :HD]
    xshi[...] = o_s[:, HD:]
    xtlo[...] = o_t[:, :HD]
    xthi[...] = o_t[:, HD:]


def _layer_tc(as_lo, as_hi, at_lo, at_hi, deg_s, deg_t, ws, bs, wt, bt, mwlo, mwhi, mb):
    blk = lambda i: (i, 0)
    full = lambda i: (0, 0)
    return pl.pallas_call(
        _layer_tc_body,
        grid=(N_BLOCKS,),
        in_specs=[
            pl.BlockSpec((R_BLK, HD), blk), pl.BlockSpec((R_BLK, HD), blk),
            pl.BlockSpec((R_BLK, HD), blk), pl.BlockSpec((R_BLK, HD), blk),
            pl.BlockSpec((R_BLK, 16), blk), pl.BlockSpec((R_BLK, 16), blk),
            pl.BlockSpec((D, D), full), pl.BlockSpec((1, D), full),
            pl.BlockSpec((D, D), full), pl.BlockSpec((1, D), full),
            pl.BlockSpec((D, D), full), pl.BlockSpec((D, D), full),
            pl.BlockSpec((1, D), full),
        ],
        out_specs=[
            pl.BlockSpec((R_BLK, HD), blk), pl.BlockSpec((R_BLK, HD), blk),
            pl.BlockSpec((R_BLK, HD), blk), pl.BlockSpec((R_BLK, HD), blk),
        ],
        out_shape=[jax.ShapeDtypeStruct((N_NODES, HD), jnp.float32)] * 4,
    )(as_lo, as_hi, at_lo, at_hi, deg_s, deg_t, ws, bs, wt, bt, mwlo, mwhi, mb)


def _mix_pred_tc_body(aslo, ashi, atlo, athi, degs, degt, ws, bs, wt, bt,
                      mwlo, mwhi, mb, ue, u1lo, u1hi, pw, pb, pu, pi):
    a_s = jnp.concatenate([aslo[...], ashi[...]], axis=1)
    a_t = jnp.concatenate([atlo[...], athi[...]], axis=1)
    inv_s = 1.0 / jnp.maximum(degs[...][:, 0:1], 1.0)
    inv_t = 1.0 / jnp.maximum(degt[...][:, 0:1], 1.0)
    c_s = jnp.dot(a_s * inv_s, ws[...], preferred_element_type=jnp.float32) + bs[...]
    c_t = jnp.dot(a_t * inv_t, wt[...], preferred_element_type=jnp.float32) + bt[...]
    u2 = (jnp.dot(c_s, mwlo[...], preferred_element_type=jnp.float32)
          + jnp.dot(c_t, mwhi[...], preferred_element_type=jnp.float32) + mb[...])
    s_blk = jnp.concatenate([ue[...], u1lo[...], u1hi[...], u2], axis=1)
    pwa = pw[...]
    pu[...] = jnp.dot(s_blk, pwa[0:3 * D, :], preferred_element_type=jnp.float32) + pb[...]
    pi[...] = jnp.dot(s_blk, pwa[3 * D:6 * D, :], preferred_element_type=jnp.float32)


def _mix_pred_tc(as_lo, as_hi, at_lo, at_hi, deg_s, deg_t, ws, bs, wt, bt,
                 mwlo, mwhi, mb, ue, u1lo, u1hi, pw, pb):
    blk = lambda i: (i, 0)
    full = lambda i: (0, 0)
    return pl.pallas_call(
        _mix_pred_tc_body,
        grid=(U_BLOCKS,),
        in_specs=[
            pl.BlockSpec((R_BLK, HD), blk), pl.BlockSpec((R_BLK, HD), blk),
            pl.BlockSpec((R_BLK, HD), blk), pl.BlockSpec((R_BLK, HD), blk),
            pl.BlockSpec((R_BLK, 16), blk), pl.BlockSpec((R_BLK, 16), blk),
            pl.BlockSpec((D, D), full), pl.BlockSpec((1, D), full),
            pl.BlockSpec((D, D), full), pl.BlockSpec((1, D), full),
            pl.BlockSpec((D, D), full), pl.BlockSpec((D, D), full),
            pl.BlockSpec((1, D), full),
            pl.BlockSpec((R_BLK, D), blk),
            pl.BlockSpec((R_BLK, HD), blk), pl.BlockSpec((R_BLK, HD), blk),
            pl.BlockSpec((6 * D, 1), full), pl.BlockSpec((1, 1), full),
        ],
        out_specs=[pl.BlockSpec((R_BLK, 1), blk), pl.BlockSpec((R_BLK, 1), blk)],
        out_shape=[jax.ShapeDtypeStruct((N_USERS, 1), jnp.float32)] * 2,
    )(as_lo, as_hi, at_lo, at_hi, deg_s, deg_t, ws, bs, wt, bt, mwlo, mwhi, mb,
      ue, u1lo, u1hi, pw, pb)


LINKS_PER_TILE = B_LINKS // (NC * NS)


@functools.partial(
    pl.kernel,
    mesh=_mesh,
    compiler_params=_sc_params,
    out_type=jax.ShapeDtypeStruct((B_LINKS,), jnp.float32),
    scratch_types=[
        pltpu.VMEM((N_USERS,), jnp.float32),
        pltpu.VMEM((N_USERS,), jnp.float32),
        pltpu.VMEM((LINKS_PER_TILE,), jnp.int32),
        pltpu.VMEM((LINKS_PER_TILE,), jnp.int32),
        pltpu.VMEM((LINKS_PER_TILE,), jnp.float32),
    ],
)
def _head_sc(pu_hbm, pi_hbm, l0_hbm, l1_hbm, out_hbm, pu_v, pi_v, l0_v, l1_v, o_v):
    c = lax.axis_index("c")
    s = lax.axis_index("s")
    wid = s * NC + c
    base = wid * LINKS_PER_TILE
    pltpu.sync_copy(pu_hbm, pu_v)
    pltpu.sync_copy(pi_hbm, pi_v)
    pltpu.sync_copy(l0_hbm.at[pl.ds(base, LINKS_PER_TILE)], l0_v)
    pltpu.sync_copy(l1_hbm.at[pl.ds(base, LINKS_PER_TILE)], l1_v)

    def body(k, carry):
        i0 = l0_v[pl.ds(k * 16, 16)]
        i1 = l1_v[pl.ds(k * 16, 16)]
        g0 = plsc.load_gather(pu_v, [i0])
        g1 = plsc.load_gather(pi_v, [i1])
        z = g0 + g1
        z = jnp.where(z >= 0.0, z, 0.01 * z)
        o_v[pl.ds(k * 16, 16)] = 1.0 / (1.0 + jnp.exp(-z))
        return carry

    lax.fori_loop(0, LINKS_PER_TILE // 16, body, 0)
    pltpu.sync_copy(o_v, out_hbm.at[pl.ds(base, LINKS_PER_TILE)])


def kernel(source_edge_index, target_edge_index, link, user_emb, src_item_emb,
           tgt_item_emb, src_conv_W, src_conv_b, tgt_conv_W, tgt_conv_b,
           mix_W, mix_b, pred_W, pred_b):
    f32 = jnp.float32
    s_src = source_edge_index[0].reshape(NSUB, CHUNK)
    d_src = source_edge_index[1].reshape(NSUB, CHUNK)
    s_tgt = target_edge_index[0].reshape(NSUB, CHUNK)
    d_tgt = target_edge_index[1].reshape(NSUB, CHUNK)
    pk_s = jnp.stack([s_src, d_src], axis=1)
    pk_t = jnp.stack([s_tgt, d_tgt], axis=1)
    l0 = link[0]
    l1 = link[1]

    xs_lo = jnp.concatenate([user_emb[:, :HD], src_item_emb[:, :HD]], axis=0)
    xs_hi = jnp.concatenate([user_emb[:, HD:], src_item_emb[:, HD:]], axis=0)
    xt_lo = jnp.concatenate([user_emb[:, :HD], tgt_item_emb[:, :HD]], axis=0)
    xt_hi = jnp.concatenate([user_emb[:, HD:], tgt_item_emb[:, HD:]], axis=0)

    zc = jnp.zeros((ZROWS, HD), f32)
    zc16 = jnp.zeros((ROWS_PER_TILE, 16), f32)

    deg_s, deg_t = _deg_sc(d_src, d_tgt, zc16)

    ws0, ws1 = src_conv_W[0], src_conv_W[1]
    wt0, wt1 = tgt_conv_W[0], tgt_conv_W[1]
    bs0 = src_conv_b[0].reshape(1, D)
    bs1 = src_conv_b[1].reshape(1, D)
    bt0 = tgt_conv_b[0].reshape(1, D)
    bt1 = tgt_conv_b[1].reshape(1, D)
    mw0lo, mw0hi = mix_W[0][:D], mix_W[0][D:]
    mw1lo, mw1hi = mix_W[1][:D], mix_W[1][D:]
    mb0 = mix_b[0].reshape(1, D)
    mb1 = mix_b[1].reshape(1, D)

    as_lo, as_hi = _conv_sc(pk_s, xs_lo, xs_hi, zc, deg_s)
    at_lo, at_hi = _conv_sc(pk_t, xt_lo, xt_hi, zc, deg_s)
    x1s_lo, x1s_hi, x1t_lo, x1t_hi = _layer_tc(
        as_lo, as_hi, at_lo, at_hi, deg_s, deg_t,
        ws0, bs0, wt0, bt0, mw0lo, mw0hi, mb0)

    a2s_lo, a2s_hi = _conv_sc(pk_s, x1s_lo, x1s_hi, zc, deg_s)
    a2t_lo, a2t_hi = _conv_sc(pk_t, x1t_lo, x1t_hi, zc, deg_s)
    pu, pi = _mix_pred_tc(a2s_lo, a2s_hi, a2t_lo, a2t_hi, deg_s, deg_t,
                          ws1, bs1, wt1, bt1, mw1lo, mw1hi, mb1,
                          user_emb, x1s_lo, x1s_hi, pred_W, pred_b.reshape(1, 1))

    out = _head_sc(pu.reshape(N_USERS), pi.reshape(N_USERS), l0, l1)
    return out.reshape(B_LINKS, 1)


# speedup vs baseline: 9.6225x; 1.0168x over previous
import functools

import jax
import jax.numpy as jnp
from jax import lax
from jax.experimental import pallas as pl
from jax.experimental.pallas import tpu as pltpu
from jax.experimental.pallas import tpu_sc as plsc

N_USERS = 20000
N_NODES = 50000
D = 64
HD = 32
E_EDGES = 800000
B_LINKS = 16384

NC = 2
NS = 16
CHUNK = 128
NSUB = E_EDGES // CHUNK
SUB_PER_TILE = -(-NSUB // NS)
ROWS_PER_TILE = N_NODES // NS
ZROWS = 3125

_mesh = plsc.VectorSubcoreMesh(core_axis_name="c", subcore_axis_name="s")
_sc_params = pltpu.CompilerParams(use_tc_tiling_on_sc=False,
                                  needs_layout_passes=False)


GRP = 16
GROUPS_PER_TILE = -(-SUB_PER_TILE // GRP)
GPAIRS = -(-GROUPS_PER_TILE // 2)


def _edge_loop(src2d, dst2d, x_hbm, acc, isbufs, idbufs, isems, rbufs, rsems,
               tid):
    my0 = tid * SUB_PER_TILE
    myend = jnp.minimum(my0 + SUB_PER_TILE, NSUB)

    def is_full(c0):
        return (c0 + GRP) <= myend

    def stage(c0, p):
        pltpu.async_copy(src2d.at[pl.ds(c0, GRP)], isbufs[p], isems[p])
        pltpu.async_copy(dst2d.at[pl.ds(c0, GRP)], idbufs[p], isems[p])

    def gather(isb, b):
        return pltpu.async_copy(x_hbm.at[isb.at[b]], rbufs[b % 2], rsems[b % 2])

    def full_group(g, p):
        c0 = my0 + g * GRP
        isb = isbufs[p]
        idb = idbufs[p]
        pltpu.make_async_copy(src2d.at[pl.ds(c0, GRP)], isb, isems[p]).wait()
        pltpu.make_async_copy(dst2d.at[pl.ds(c0, GRP)], idb, isems[p]).wait()

        @pl.when(is_full(my0 + (g + 1) * GRP))
        def _():
            stage(my0 + (g + 1) * GRP, 1 - p)

        h = gather(isb, 0)
        for b in range(GRP):
            if b + 1 < GRP:
                h_next = gather(isb, b + 1)
            h.wait()
            pltpu.sync_copy(rbufs[b % 2], acc.at[idb.at[b]], add=True)
            if b + 1 < GRP:
                h = h_next

    def tail_group(g, p):
        c0 = my0 + g * GRP
        isb = isbufs[p]
        idb = idbufs[p]

        def tail(b, carry2):
            sub = c0 + b

            @pl.when(sub < myend)
            def _():
                pltpu.sync_copy(src2d.at[pl.ds(sub, 1)], isb.at[pl.ds(0, 1)])
                pltpu.sync_copy(dst2d.at[pl.ds(sub, 1)], idb.at[pl.ds(0, 1)])
                pltpu.async_copy(x_hbm.at[isb.at[0]], rbufs[0], rsems[0]).wait()
                pltpu.sync_copy(rbufs[0], acc.at[idb.at[0]], add=True)

            return carry2

        lax.fori_loop(0, GRP, tail, 0)

    def body(k, carry):
        for p in (0, 1):
            g = 2 * k + p
            if 2 * GPAIRS > GROUPS_PER_TILE and p == 1:
                guard_g = g < GROUPS_PER_TILE
            else:
                guard_g = None
            c0 = my0 + g * GRP
            cond_full = is_full(c0)
            if guard_g is not None:
                cond_full = jnp.logical_and(guard_g, cond_full)
                cond_tail = jnp.logical_and(guard_g, jnp.logical_not(is_full(c0)))
            else:
                cond_tail = jnp.logical_not(is_full(c0))

            @pl.when(cond_full)
            def _(g=g, p=p):
                full_group(g, p)

            @pl.when(cond_tail)
            def _(g=g, p=p):
                tail_group(g, p)

        return carry

    @pl.when(is_full(my0))
    def _():
        stage(my0, 0)

    lax.fori_loop(0, GPAIRS, body, 0)


def _one_graph(src2d, dst2d, xlo, xhi, outlo, outhi, zc, acc, isbufs, idbufs,
               isems, rbufs, rsems, c, s):
    pltpu.sync_copy(zc, acc.at[pl.ds(s * ROWS_PER_TILE, ZROWS)])
    plsc.subcore_barrier()

    @pl.when(c == 0)
    def _():
        _edge_loop(src2d, dst2d, xlo, acc, isbufs, idbufs, isems, rbufs,
                   rsems, s)

    @pl.when(c == 1)
    def _():
        _edge_loop(src2d, dst2d, xhi, acc, isbufs, idbufs, isems, rbufs,
                   rsems, s)

    plsc.subcore_barrier()
    r0 = s * ROWS_PER_TILE

    @pl.when(c == 0)
    def _():
        pltpu.sync_copy(acc.at[pl.ds(r0, ROWS_PER_TILE)], outlo.at[pl.ds(r0, ROWS_PER_TILE)])

    @pl.when(c == 1)
    def _():
        pltpu.sync_copy(acc.at[pl.ds(r0, ROWS_PER_TILE)], outhi.at[pl.ds(r0, ROWS_PER_TILE)])


@functools.partial(
    pl.kernel,
    mesh=_mesh,
    compiler_params=_sc_params,
    out_type=[
        jax.ShapeDtypeStruct((N_NODES, HD), jnp.float32),
        jax.ShapeDtypeStruct((N_NODES, HD), jnp.float32),
    ],
    scratch_types=[
        pltpu.VMEM((GRP, CHUNK), jnp.int32),
        pltpu.VMEM((GRP, CHUNK), jnp.int32),
        pltpu.VMEM((GRP, CHUNK), jnp.int32),
        pltpu.VMEM((GRP, CHUNK), jnp.int32),
        pltpu.VMEM((CHUNK, HD), jnp.float32),
        pltpu.VMEM((CHUNK, HD), jnp.float32),
        pltpu.VMEM_SHARED((N_NODES, HD), jnp.float32),
        pltpu.SemaphoreType.DMA,
        pltpu.SemaphoreType.DMA,
        pltpu.SemaphoreType.DMA,
        pltpu.SemaphoreType.DMA,
    ],
)
def _conv_sc(src2d, dst2d, x_lo, x_hi, zc, ord_dep,
             a_lo, a_hi,
             isbuf0, isbuf1, idbuf0, idbuf1, rows_a, rows_b, acc,
             isem0, isem1, rsem_a, rsem_b):
    c = lax.axis_index("c")
    s = lax.axis_index("s")
    isbufs = (isbuf0, isbuf1)
    idbufs = (idbuf0, idbuf1)
    isems = (isem0, isem1)
    rbufs = (rows_a, rows_b)
    rsems = (rsem_a, rsem_b)
    _one_graph(src2d, dst2d, x_lo, x_hi, a_lo, a_hi, zc, acc, isbufs, idbufs,
               isems, rbufs, rsems, c, s)


def _deg_loop(dst2d, acc, idg, ones_v):
    s = lax.axis_index("s")
    my0 = s * SUB_PER_TILE
    myend = jnp.minimum(my0 + SUB_PER_TILE, NSUB)

    def body(g, carry):
        c0 = my0 + g * GRP
        full = (c0 + GRP) <= myend

        @pl.when(full)
        def _():
            pltpu.sync_copy(dst2d.at[pl.ds(c0, GRP)], idg)
            for b in range(GRP):
                pltpu.sync_copy(ones_v, acc.at[idg.at[b]], add=True)

        @pl.when(jnp.logical_not(full))
        def _():
            def tail(b, carry2):
                sub = c0 + b

                @pl.when(sub < myend)
                def _():
                    pltpu.sync_copy(dst2d.at[pl.ds(sub, 1)], idg.at[pl.ds(0, 1)])
                    pltpu.sync_copy(ones_v, acc.at[idg.at[0]], add=True)

                return carry2

            lax.fori_loop(0, GRP, tail, 0)

        return carry

    lax.fori_loop(0, GROUPS_PER_TILE, body, 0)


@functools.partial(
    pl.kernel,
    mesh=_mesh,
    compiler_params=_sc_params,
    out_type=[
        jax.ShapeDtypeStruct((N_NODES, 16), jnp.float32),
        jax.ShapeDtypeStruct((N_NODES, 16), jnp.float32),
    ],
    scratch_types=[
        pltpu.VMEM((GRP, CHUNK), jnp.int32),
        pltpu.VMEM((CHUNK, 16), jnp.float32),
        pltpu.VMEM_SHARED((N_NODES, 16), jnp.float32),
    ],
)
def _deg_sc(dst_s, dst_t, zc16, deg_s, deg_t, idg, ones_v, acc):
    c = lax.axis_index("c")
    s = lax.axis_index("s")

    one = jnp.ones((16,), jnp.float32)
    for r in range(CHUNK):
        ones_v[r, 0:16] = one

    pltpu.sync_copy(zc16, acc.at[pl.ds(s * ROWS_PER_TILE, ROWS_PER_TILE)])
    plsc.subcore_barrier()

    @pl.when(c == 0)
    def _():
        _deg_loop(dst_s, acc, idg, ones_v)

    @pl.when(c == 1)
    def _():
        _deg_loop(dst_t, acc, idg, ones_v)

    plsc.subcore_barrier()
    r0 = s * ROWS_PER_TILE

    @pl.when(c == 0)
    def _():
        pltpu.sync_copy(acc.at[pl.ds(r0, ROWS_PER_TILE)], deg_s.at[pl.ds(r0, ROWS_PER_TILE)])

    @pl.when(c == 1)
    def _():
        pltpu.sync_copy(acc.at[pl.ds(r0, ROWS_PER_TILE)], deg_t.at[pl.ds(r0, ROWS_PER_TILE)])


R_BLK = 5000
N_BLOCKS = N_NODES // R_BLK
U_BLOCKS = N_USERS // R_BLK


def _layer_tc_body(aslo, ashi, atlo, athi, degs, degt, ws, bs, wt, bt,
                   mwlo, mwhi, mb, xslo, xshi, xtlo, xthi):
    i = pl.program_id(0)
    a_s = jnp.concatenate([aslo[...], ashi[...]], axis=1)
    a_t = jnp.concatenate([atlo[...], athi[...]], axis=1)
    inv_s = 1.0 / jnp.maximum(degs[...][:, 0:1], 1.0)
    inv_t = 1.0 / jnp.maximum(degt[...][:, 0:1], 1.0)
    c_s = jnp.dot(a_s * inv_s, ws[...], preferred_element_type=jnp.float32) + bs[...]
    c_t = jnp.dot(a_t * inv_t, wt[...], preferred_element_type=jnp.float32) + bt[...]
    u = (jnp.dot(c_s, mwlo[...], preferred_element_type=jnp.float32)
         + jnp.dot(c_t, mwhi[...], preferred_element_type=jnp.float32) + mb[...])
    is_user = (i < U_BLOCKS)
    o_s = jnp.where(is_user, u, c_s)
    o_t = jnp.where(is_user, u, c_t)
    xslo[...] = o_s[:, :HD]
    xshi[...] = o_s[:, HD:]
    xtlo[...] = o_t[:, :HD]
    xthi[...] = o_t[:, HD:]


def _layer_tc(as_lo, as_hi, at_lo, at_hi, deg_s, deg_t, ws, bs, wt, bt, mwlo, mwhi, mb):
    blk = lambda i: (i, 0)
    full = lambda i: (0, 0)
    return pl.pallas_call(
        _layer_tc_body,
        grid=(N_BLOCKS,),
        in_specs=[
            pl.BlockSpec((R_BLK, HD), blk), pl.BlockSpec((R_BLK, HD), blk),
            pl.BlockSpec((R_BLK, HD), blk), pl.BlockSpec((R_BLK, HD), blk),
            pl.BlockSpec((R_BLK, 16), blk), pl.BlockSpec((R_BLK, 16), blk),
            pl.BlockSpec((D, D), full), pl.BlockSpec((1, D), full),
            pl.BlockSpec((D, D), full), pl.BlockSpec((1, D), full),
            pl.BlockSpec((D, D), full), pl.BlockSpec((D, D), full),
            pl.BlockSpec((1, D), full),
        ],
        out_specs=[
            pl.BlockSpec((R_BLK, HD), blk), pl.BlockSpec((R_BLK, HD), blk),
            pl.BlockSpec((R_BLK, HD), blk), pl.BlockSpec((R_BLK, HD), blk),
        ],
        out_shape=[jax.ShapeDtypeStruct((N_NODES, HD), jnp.float32)] * 4,
    )(as_lo, as_hi, at_lo, at_hi, deg_s, deg_t, ws, bs, wt, bt, mwlo, mwhi, mb)


def _mix_pred_tc_body(aslo, ashi, atlo, athi, degs, degt, ws, bs, wt, bt,
                      mwlo, mwhi, mb, ue, u1lo, u1hi, pw, pb, pu, pi):
    a_s = jnp.concatenate([aslo[...], ashi[...]], axis=1)
    a_t = jnp.concatenate([atlo[...], athi[...]], axis=1)
    inv_s = 1.0 / jnp.maximum(degs[...][:, 0:1], 1.0)
    inv_t = 1.0 / jnp.maximum(degt[...][:, 0:1], 1.0)
    c_s = jnp.dot(a_s * inv_s, ws[...], preferred_element_type=jnp.float32) + bs[...]
    c_t = jnp.dot(a_t * inv_t, wt[...], preferred_element_type=jnp.float32) + bt[...]
    u2 = (jnp.dot(c_s, mwlo[...], preferred_element_type=jnp.float32)
          + jnp.dot(c_t, mwhi[...], preferred_element_type=jnp.float32) + mb[...])
    s_blk = jnp.concatenate([ue[...], u1lo[...], u1hi[...], u2], axis=1)
    pwa = pw[...]
    pu[...] = jnp.dot(s_blk, pwa[0:3 * D, :], preferred_element_type=jnp.float32) + pb[...]
    pi[...] = jnp.dot(s_blk, pwa[3 * D:6 * D, :], preferred_element_type=jnp.float32)


def _mix_pred_tc(as_lo, as_hi, at_lo, at_hi, deg_s, deg_t, ws, bs, wt, bt,
                 mwlo, mwhi, mb, ue, u1lo, u1hi, pw, pb):
    blk = lambda i: (i, 0)
    full = lambda i: (0, 0)
    return pl.pallas_call(
        _mix_pred_tc_body,
        grid=(U_BLOCKS,),
        in_specs=[
            pl.BlockSpec((R_BLK, HD), blk), pl.BlockSpec((R_BLK, HD), blk),
            pl.BlockSpec((R_BLK, HD), blk), pl.BlockSpec((R_BLK, HD), blk),
            pl.BlockSpec((R_BLK, 16), blk), pl.BlockSpec((R_BLK, 16), blk),
            pl.BlockSpec((D, D), full), pl.BlockSpec((1, D), full),
            pl.BlockSpec((D, D), full), pl.BlockSpec((1, D), full),
            pl.BlockSpec((D, D), full), pl.BlockSpec((D, D), full),
            pl.BlockSpec((1, D), full),
            pl.BlockSpec((R_BLK, D), blk),
            pl.BlockSpec((R_BLK, HD), blk), pl.BlockSpec((R_BLK, HD), blk),
            pl.BlockSpec((6 * D, 1), full), pl.BlockSpec((1, 1), full),
        ],
        out_specs=[pl.BlockSpec((R_BLK, 1), blk), pl.BlockSpec((R_BLK, 1), blk)],
        out_shape=[jax.ShapeDtypeStruct((N_USERS, 1), jnp.float32)] * 2,
    )(as_lo, as_hi, at_lo, at_hi, deg_s, deg_t, ws, bs, wt, bt, mwlo, mwhi, mb,
      ue, u1lo, u1hi, pw, pb)


LINKS_PER_TILE = B_LINKS // (NC * NS)


@functools.partial(
    pl.kernel,
    mesh=_mesh,
    compiler_params=_sc_params,
    out_type=jax.ShapeDtypeStruct((B_LINKS,), jnp.float32),
    scratch_types=[
        pltpu.VMEM((N_USERS,), jnp.float32),
        pltpu.VMEM((N_USERS,), jnp.float32),
        pltpu.VMEM((LINKS_PER_TILE,), jnp.int32),
        pltpu.VMEM((LINKS_PER_TILE,), jnp.int32),
        pltpu.VMEM((LINKS_PER_TILE,), jnp.float32),
    ],
)
def _head_sc(pu_hbm, pi_hbm, l0_hbm, l1_hbm, out_hbm, pu_v, pi_v, l0_v, l1_v, o_v):
    c = lax.axis_index("c")
    s = lax.axis_index("s")
    wid = s * NC + c
    base = wid * LINKS_PER_TILE
    pltpu.sync_copy(pu_hbm, pu_v)
    pltpu.sync_copy(pi_hbm, pi_v)
    pltpu.sync_copy(l0_hbm.at[pl.ds(base, LINKS_PER_TILE)], l0_v)
    pltpu.sync_copy(l1_hbm.at[pl.ds(base, LINKS_PER_TILE)], l1_v)

    def body(k, carry):
        i0 = l0_v[pl.ds(k * 16, 16)]
        i1 = l1_v[pl.ds(k * 16, 16)]
        g0 = plsc.load_gather(pu_v, [i0])
        g1 = plsc.load_gather(pi_v, [i1])
        z = g0 + g1
        z = jnp.where(z >= 0.0, z, 0.01 * z)
        o_v[pl.ds(k * 16, 16)] = 1.0 / (1.0 + jnp.exp(-z))
        return carry

    lax.fori_loop(0, LINKS_PER_TILE // 16, body, 0)
    pltpu.sync_copy(o_v, out_hbm.at[pl.ds(base, LINKS_PER_TILE)])


def kernel(source_edge_index, target_edge_index, link, user_emb, src_item_emb,
           tgt_item_emb, src_conv_W, src_conv_b, tgt_conv_W, tgt_conv_b,
           mix_W, mix_b, pred_W, pred_b):
    f32 = jnp.float32
    s_src = source_edge_index[0].reshape(NSUB, CHUNK)
    d_src = source_edge_index[1].reshape(NSUB, CHUNK)
    s_tgt = target_edge_index[0].reshape(NSUB, CHUNK)
    d_tgt = target_edge_index[1].reshape(NSUB, CHUNK)
    l0 = link[0]
    l1 = link[1]

    xs_lo = jnp.concatenate([user_emb[:, :HD], src_item_emb[:, :HD]], axis=0)
    xs_hi = jnp.concatenate([user_emb[:, HD:], src_item_emb[:, HD:]], axis=0)
    xt_lo = jnp.concatenate([user_emb[:, :HD], tgt_item_emb[:, :HD]], axis=0)
    xt_hi = jnp.concatenate([user_emb[:, HD:], tgt_item_emb[:, HD:]], axis=0)

    zc = jnp.zeros((ZROWS, HD), f32)
    zc16 = jnp.zeros((ROWS_PER_TILE, 16), f32)

    deg_s, deg_t = _deg_sc(d_src, d_tgt, zc16)

    ws0, ws1 = src_conv_W[0], src_conv_W[1]
    wt0, wt1 = tgt_conv_W[0], tgt_conv_W[1]
    bs0 = src_conv_b[0].reshape(1, D)
    bs1 = src_conv_b[1].reshape(1, D)
    bt0 = tgt_conv_b[0].reshape(1, D)
    bt1 = tgt_conv_b[1].reshape(1, D)
    mw0lo, mw0hi = mix_W[0][:D], mix_W[0][D:]
    mw1lo, mw1hi = mix_W[1][:D], mix_W[1][D:]
    mb0 = mix_b[0].reshape(1, D)
    mb1 = mix_b[1].reshape(1, D)

    as_lo, as_hi = _conv_sc(s_src, d_src, xs_lo, xs_hi, zc, deg_s)
    at_lo, at_hi = _conv_sc(s_tgt, d_tgt, xt_lo, xt_hi, zc, deg_s)
    x1s_lo, x1s_hi, x1t_lo, x1t_hi = _layer_tc(
        as_lo, as_hi, at_lo, at_hi, deg_s, deg_t,
        ws0, bs0, wt0, bt0, mw0lo, mw0hi, mb0)

    a2s_lo, a2s_hi = _conv_sc(s_src, d_src, x1s_lo, x1s_hi, zc, deg_s)
    a2t_lo, a2t_hi = _conv_sc(s_tgt, d_tgt, x1t_lo, x1t_hi, zc, deg_s)
    pu, pi = _mix_pred_tc(a2s_lo, a2s_hi, a2t_lo, a2t_hi, deg_s, deg_t,
                          ws1, bs1, wt1, bt1, mw1lo, mw1hi, mb1,
                          user_emb, x1s_lo, x1s_hi, pred_W, pred_b.reshape(1, 1))

    out = _head_sc(pu.reshape(N_USERS), pi.reshape(N_USERS), l0, l1)
    return out.reshape(B_LINKS, 1)


# speedup vs baseline: 11.9281x; 1.2396x over previous
import functools

import jax
import jax.numpy as jnp
from jax import lax
from jax.experimental import pallas as pl
from jax.experimental.pallas import tpu as pltpu
from jax.experimental.pallas import tpu_sc as plsc

N_USERS = 20000
N_NODES = 50000
D = 64
HD = 32
E_EDGES = 800000
B_LINKS = 16384

NC = 2
NS = 16
CHUNK = 128
NSUB = E_EDGES // CHUNK
SUB_PER_TILE = -(-NSUB // NS)
ROWS_PER_TILE = N_NODES // NS
ZROWS = 3125

_mesh = plsc.VectorSubcoreMesh(core_axis_name="c", subcore_axis_name="s")
_sc_params = pltpu.CompilerParams(use_tc_tiling_on_sc=False,
                                  needs_layout_passes=False)


GRP = 16
GROUPS_PER_TILE = -(-SUB_PER_TILE // GRP)
GPAIRS = -(-GROUPS_PER_TILE // 2)
NBUF = 4


def _edge_loop(src2d, dst2d, x_hbm, acc, isbufs, idbufs, isems, rbufs, rsems,
               tid):
    my0 = tid * SUB_PER_TILE
    myend = jnp.minimum(my0 + SUB_PER_TILE, NSUB)

    def is_full(c0):
        return (c0 + GRP) <= myend

    def stage(c0, p):
        pltpu.async_copy(src2d.at[pl.ds(c0, GRP)], isbufs[p], isems[p])
        pltpu.async_copy(dst2d.at[pl.ds(c0, GRP)], idbufs[p], isems[p])

    def gather(isb, b):
        return pltpu.async_copy(x_hbm.at[isb.at[b]], rbufs[b % NBUF],
                                rsems[b % NBUF])

    def full_group(g, p):
        c0 = my0 + g * GRP
        isb = isbufs[p]
        idb = idbufs[p]
        pltpu.make_async_copy(src2d.at[pl.ds(c0, GRP)], isb, isems[p]).wait()
        pltpu.make_async_copy(dst2d.at[pl.ds(c0, GRP)], idb, isems[p]).wait()

        @pl.when(is_full(my0 + (g + 1) * GRP))
        def _():
            stage(my0 + (g + 1) * GRP, 1 - p)

        hs = {}
        for j in range(NBUF - 1):
            hs[j] = gather(isb, j)
        for b in range(GRP):
            if b + NBUF - 1 < GRP:
                hs[b + NBUF - 1] = gather(isb, b + NBUF - 1)
            hs.pop(b).wait()
            pltpu.sync_copy(rbufs[b % NBUF], acc.at[idb.at[b]], add=True)

    def tail_group(g, p):
        c0 = my0 + g * GRP
        isb = isbufs[p]
        idb = idbufs[p]

        def tail(b, carry2):
            sub = c0 + b

            @pl.when(sub < myend)
            def _():
                pltpu.sync_copy(src2d.at[pl.ds(sub, 1)], isb.at[pl.ds(0, 1)])
                pltpu.sync_copy(dst2d.at[pl.ds(sub, 1)], idb.at[pl.ds(0, 1)])
                pltpu.async_copy(x_hbm.at[isb.at[0]], rbufs[0], rsems[0]).wait()
                pltpu.sync_copy(rbufs[0], acc.at[idb.at[0]], add=True)

            return carry2

        lax.fori_loop(0, GRP, tail, 0)

    def body(k, carry):
        for p in (0, 1):
            g = 2 * k + p
            if 2 * GPAIRS > GROUPS_PER_TILE and p == 1:
                guard_g = g < GROUPS_PER_TILE
            else:
                guard_g = None
            c0 = my0 + g * GRP
            cond_full = is_full(c0)
            if guard_g is not None:
                cond_full = jnp.logical_and(guard_g, cond_full)
                cond_tail = jnp.logical_and(guard_g, jnp.logical_not(is_full(c0)))
            else:
                cond_tail = jnp.logical_not(is_full(c0))

            @pl.when(cond_full)
            def _(g=g, p=p):
                full_group(g, p)

            @pl.when(cond_tail)
            def _(g=g, p=p):
                tail_group(g, p)

        return carry

    @pl.when(is_full(my0))
    def _():
        stage(my0, 0)

    lax.fori_loop(0, GPAIRS, body, 0)


def _one_graph(src2d, dst2d, xlo, xhi, outlo, outhi, zc, acc, isbufs, idbufs,
               isems, rbufs, rsems, c, s):
    pltpu.sync_copy(zc, acc.at[pl.ds(s * ROWS_PER_TILE, ZROWS)])
    plsc.subcore_barrier()

    @pl.when(c == 0)
    def _():
        _edge_loop(src2d, dst2d, xlo, acc, isbufs, idbufs, isems, rbufs,
                   rsems, s)

    @pl.when(c == 1)
    def _():
        _edge_loop(src2d, dst2d, xhi, acc, isbufs, idbufs, isems, rbufs,
                   rsems, s)

    plsc.subcore_barrier()
    r0 = s * ROWS_PER_TILE

    @pl.when(c == 0)
    def _():
        pltpu.sync_copy(acc.at[pl.ds(r0, ROWS_PER_TILE)], outlo.at[pl.ds(r0, ROWS_PER_TILE)])

    @pl.when(c == 1)
    def _():
        pltpu.sync_copy(acc.at[pl.ds(r0, ROWS_PER_TILE)], outhi.at[pl.ds(r0, ROWS_PER_TILE)])


@functools.partial(
    pl.kernel,
    mesh=_mesh,
    compiler_params=_sc_params,
    out_type=[
        jax.ShapeDtypeStruct((N_NODES, HD), jnp.float32),
        jax.ShapeDtypeStruct((N_NODES, HD), jnp.float32),
    ],
    scratch_types=[
        pltpu.VMEM((GRP, CHUNK), jnp.int32),
        pltpu.VMEM((GRP, CHUNK), jnp.int32),
        pltpu.VMEM((GRP, CHUNK), jnp.int32),
        pltpu.VMEM((GRP, CHUNK), jnp.int32),
        pltpu.VMEM((CHUNK, HD), jnp.float32),
        pltpu.VMEM((CHUNK, HD), jnp.float32),
        pltpu.VMEM((CHUNK, HD), jnp.float32),
        pltpu.VMEM((CHUNK, HD), jnp.float32),
        pltpu.VMEM_SHARED((N_NODES, HD), jnp.float32),
        pltpu.SemaphoreType.DMA,
        pltpu.SemaphoreType.DMA,
        pltpu.SemaphoreType.DMA,
        pltpu.SemaphoreType.DMA,
        pltpu.SemaphoreType.DMA,
        pltpu.SemaphoreType.DMA,
    ],
)
def _conv_sc(src2d, dst2d, x_lo, x_hi, zc, ord_dep,
             a_lo, a_hi,
             isbuf0, isbuf1, idbuf0, idbuf1, rows_a, rows_b, rows_c, rows_d,
             acc, isem0, isem1, rsem_a, rsem_b, rsem_c, rsem_d):
    c = lax.axis_index("c")
    s = lax.axis_index("s")
    isbufs = (isbuf0, isbuf1)
    idbufs = (idbuf0, idbuf1)
    isems = (isem0, isem1)
    rbufs = (rows_a, rows_b, rows_c, rows_d)
    rsems = (rsem_a, rsem_b, rsem_c, rsem_d)
    _one_graph(src2d, dst2d, x_lo, x_hi, a_lo, a_hi, zc, acc, isbufs, idbufs,
               isems, rbufs, rsems, c, s)


def _deg_loop(dst2d, acc, idg, ones_v):
    s = lax.axis_index("s")
    my0 = s * SUB_PER_TILE
    myend = jnp.minimum(my0 + SUB_PER_TILE, NSUB)

    def body(g, carry):
        c0 = my0 + g * GRP
        full = (c0 + GRP) <= myend

        @pl.when(full)
        def _():
            pltpu.sync_copy(dst2d.at[pl.ds(c0, GRP)], idg)
            for b in range(GRP):
                pltpu.sync_copy(ones_v, acc.at[idg.at[b]], add=True)

        @pl.when(jnp.logical_not(full))
        def _():
            def tail(b, carry2):
                sub = c0 + b

                @pl.when(sub < myend)
                def _():
                    pltpu.sync_copy(dst2d.at[pl.ds(sub, 1)], idg.at[pl.ds(0, 1)])
                    pltpu.sync_copy(ones_v, acc.at[idg.at[0]], add=True)

                return carry2

            lax.fori_loop(0, GRP, tail, 0)

        return carry

    lax.fori_loop(0, GROUPS_PER_TILE, body, 0)


@functools.partial(
    pl.kernel,
    mesh=_mesh,
    compiler_params=_sc_params,
    out_type=[
        jax.ShapeDtypeStruct((N_NODES, 16), jnp.float32),
        jax.ShapeDtypeStruct((N_NODES, 16), jnp.float32),
    ],
    scratch_types=[
        pltpu.VMEM((GRP, CHUNK), jnp.int32),
        pltpu.VMEM((CHUNK, 16), jnp.float32),
        pltpu.VMEM_SHARED((N_NODES, 16), jnp.float32),
    ],
)
def _deg_sc(dst_s, dst_t, zc16, deg_s, deg_t, idg, ones_v, acc):
    c = lax.axis_index("c")
    s = lax.axis_index("s")

    one = jnp.ones((16,), jnp.float32)
    for r in range(CHUNK):
        ones_v[r, 0:16] = one

    pltpu.sync_copy(zc16, acc.at[pl.ds(s * ROWS_PER_TILE, ROWS_PER_TILE)])
    plsc.subcore_barrier()

    @pl.when(c == 0)
    def _():
        _deg_loop(dst_s, acc, idg, ones_v)

    @pl.when(c == 1)
    def _():
        _deg_loop(dst_t, acc, idg, ones_v)

    plsc.subcore_barrier()
    r0 = s * ROWS_PER_TILE

    @pl.when(c == 0)
    def _():
        pltpu.sync_copy(acc.at[pl.ds(r0, ROWS_PER_TILE)], deg_s.at[pl.ds(r0, ROWS_PER_TILE)])

    @pl.when(c == 1)
    def _():
        pltpu.sync_copy(acc.at[pl.ds(r0, ROWS_PER_TILE)], deg_t.at[pl.ds(r0, ROWS_PER_TILE)])


R_BLK = 5000
N_BLOCKS = N_NODES // R_BLK
U_BLOCKS = N_USERS // R_BLK


def _layer_tc_body(aslo, ashi, atlo, athi, degs, degt, ws, bs, wt, bt,
                   mwlo, mwhi, mb, xslo, xshi, xtlo, xthi):
    i = pl.program_id(0)
    a_s = jnp.concatenate([aslo[...], ashi[...]], axis=1)
    a_t = jnp.concatenate([atlo[...], athi[...]], axis=1)
    inv_s = 1.0 / jnp.maximum(degs[...][:, 0:1], 1.0)
    inv_t = 1.0 / jnp.maximum(degt[...][:, 0:1], 1.0)
    c_s = jnp.dot(a_s * inv_s, ws[...], preferred_element_type=jnp.float32) + bs[...]
    c_t = jnp.dot(a_t * inv_t, wt[...], preferred_element_type=jnp.float32) + bt[...]
    u = (jnp.dot(c_s, mwlo[...], preferred_element_type=jnp.float32)
         + jnp.dot(c_t, mwhi[...], preferred_element_type=jnp.float32) + mb[...])
    is_user = (i < U_BLOCKS)
    o_s = jnp.where(is_user, u, c_s)
    o_t = jnp.where(is_user, u, c_t)
    xslo[...] = o_s[:, :HD]
    xshi[...] = o_s[:, HD:]
    xtlo[...] = o_t[:, :HD]
    xthi[...] = o_t[:, HD:]


def _layer_tc(as_lo, as_hi, at_lo, at_hi, deg_s, deg_t, ws, bs, wt, bt, mwlo, mwhi, mb):
    blk = lambda i: (i, 0)
    full = lambda i: (0, 0)
    return pl.pallas_call(
        _layer_tc_body,
        grid=(N_BLOCKS,),
        in_specs=[
            pl.BlockSpec((R_BLK, HD), blk), pl.BlockSpec((R_BLK, HD), blk),
            pl.BlockSpec((R_BLK, HD), blk), pl.BlockSpec((R_BLK, HD), blk),
            pl.BlockSpec((R_BLK, 16), blk), pl.BlockSpec((R_BLK, 16), blk),
            pl.BlockSpec((D, D), full), pl.BlockSpec((1, D), full),
            pl.BlockSpec((D, D), full), pl.BlockSpec((1, D), full),
            pl.BlockSpec((D, D), full), pl.BlockSpec((D, D), full),
            pl.BlockSpec((1, D), full),
        ],
        out_specs=[
            pl.BlockSpec((R_BLK, HD), blk), pl.BlockSpec((R_BLK, HD), blk),
            pl.BlockSpec((R_BLK, HD), blk), pl.BlockSpec((R_BLK, HD), blk),
        ],
        out_shape=[jax.ShapeDtypeStruct((N_NODES, HD), jnp.float32)] * 4,
    )(as_lo, as_hi, at_lo, at_hi, deg_s, deg_t, ws, bs, wt, bt, mwlo, mwhi, mb)


def _mix_pred_tc_body(aslo, ashi, atlo, athi, degs, degt, ws, bs, wt, bt,
                      mwlo, mwhi, mb, ue, u1lo, u1hi, pw, pb, pu, pi):
    a_s = jnp.concatenate([aslo[...], ashi[...]], axis=1)
    a_t = jnp.concatenate([atlo[...], athi[...]], axis=1)
    inv_s = 1.0 / jnp.maximum(degs[...][:, 0:1], 1.0)
    inv_t = 1.0 / jnp.maximum(degt[...][:, 0:1], 1.0)
    c_s = jnp.dot(a_s * inv_s, ws[...], preferred_element_type=jnp.float32) + bs[...]
    c_t = jnp.dot(a_t * inv_t, wt[...], preferred_element_type=jnp.float32) + bt[...]
    u2 = (jnp.dot(c_s, mwlo[...], preferred_element_type=jnp.float32)
          + jnp.dot(c_t, mwhi[...], preferred_element_type=jnp.float32) + mb[...])
    s_blk = jnp.concatenate([ue[...], u1lo[...], u1hi[...], u2], axis=1)
    pwa = pw[...]
    pu[...] = jnp.dot(s_blk, pwa[0:3 * D, :], preferred_element_type=jnp.float32) + pb[...]
    pi[...] = jnp.dot(s_blk, pwa[3 * D:6 * D, :], preferred_element_type=jnp.float32)


def _mix_pred_tc(as_lo, as_hi, at_lo, at_hi, deg_s, deg_t, ws, bs, wt, bt,
                 mwlo, mwhi, mb, ue, u1lo, u1hi, pw, pb):
    blk = lambda i: (i, 0)
    full = lambda i: (0, 0)
    return pl.pallas_call(
        _mix_pred_tc_body,
        grid=(U_BLOCKS,),
        in_specs=[
            pl.BlockSpec((R_BLK, HD), blk), pl.BlockSpec((R_BLK, HD), blk),
            pl.BlockSpec((R_BLK, HD), blk), pl.BlockSpec((R_BLK, HD), blk),
            pl.BlockSpec((R_BLK, 16), blk), pl.BlockSpec((R_BLK, 16), blk),
            pl.BlockSpec((D, D), full), pl.BlockSpec((1, D), full),
            pl.BlockSpec((D, D), full), pl.BlockSpec((1, D), full),
            pl.BlockSpec((D, D), full), pl.BlockSpec((D, D), full),
            pl.BlockSpec((1, D), full),
            pl.BlockSpec((R_BLK, D), blk),
            pl.BlockSpec((R_BLK, HD), blk), pl.BlockSpec((R_BLK, HD), blk),
            pl.BlockSpec((6 * D, 1), full), pl.BlockSpec((1, 1), full),
        ],
        out_specs=[pl.BlockSpec((R_BLK, 1), blk), pl.BlockSpec((R_BLK, 1), blk)],
        out_shape=[jax.ShapeDtypeStruct((N_USERS, 1), jnp.float32)] * 2,
    )(as_lo, as_hi, at_lo, at_hi, deg_s, deg_t, ws, bs, wt, bt, mwlo, mwhi, mb,
      ue, u1lo, u1hi, pw, pb)


LINKS_PER_TILE = B_LINKS // (NC * NS)


@functools.partial(
    pl.kernel,
    mesh=_mesh,
    compiler_params=_sc_params,
    out_type=jax.ShapeDtypeStruct((B_LINKS,), jnp.float32),
    scratch_types=[
        pltpu.VMEM((N_USERS,), jnp.float32),
        pltpu.VMEM((N_USERS,), jnp.float32),
        pltpu.VMEM((LINKS_PER_TILE,), jnp.int32),
        pltpu.VMEM((LINKS_PER_TILE,), jnp.int32),
        pltpu.VMEM((LINKS_PER_TILE,), jnp.float32),
    ],
)
def _head_sc(pu_hbm, pi_hbm, l0_hbm, l1_hbm, out_hbm, pu_v, pi_v, l0_v, l1_v, o_v):
    c = lax.axis_index("c")
    s = lax.axis_index("s")
    wid = s * NC + c
    base = wid * LINKS_PER_TILE
    pltpu.sync_copy(pu_hbm, pu_v)
    pltpu.sync_copy(pi_hbm, pi_v)
    pltpu.sync_copy(l0_hbm.at[pl.ds(base, LINKS_PER_TILE)], l0_v)
    pltpu.sync_copy(l1_hbm.at[pl.ds(base, LINKS_PER_TILE)], l1_v)

    def body(k, carry):
        i0 = l0_v[pl.ds(k * 16, 16)]
        i1 = l1_v[pl.ds(k * 16, 16)]
        g0 = plsc.load_gather(pu_v, [i0])
        g1 = plsc.load_gather(pi_v, [i1])
        z = g0 + g1
        z = jnp.where(z >= 0.0, z, 0.01 * z)
        o_v[pl.ds(k * 16, 16)] = 1.0 / (1.0 + jnp.exp(-z))
        return carry

    lax.fori_loop(0, LINKS_PER_TILE // 16, body, 0)
    pltpu.sync_copy(o_v, out_hbm.at[pl.ds(base, LINKS_PER_TILE)])


def kernel(source_edge_index, target_edge_index, link, user_emb, src_item_emb,
           tgt_item_emb, src_conv_W, src_conv_b, tgt_conv_W, tgt_conv_b,
           mix_W, mix_b, pred_W, pred_b):
    f32 = jnp.float32
    s_src = source_edge_index[0].reshape(NSUB, CHUNK)
    d_src = source_edge_index[1].reshape(NSUB, CHUNK)
    s_tgt = target_edge_index[0].reshape(NSUB, CHUNK)
    d_tgt = target_edge_index[1].reshape(NSUB, CHUNK)
    l0 = link[0]
    l1 = link[1]

    xs_lo = jnp.concatenate([user_emb[:, :HD], src_item_emb[:, :HD]], axis=0)
    xs_hi = jnp.concatenate([user_emb[:, HD:], src_item_emb[:, HD:]], axis=0)
    xt_lo = jnp.concatenate([user_emb[:, :HD], tgt_item_emb[:, :HD]], axis=0)
    xt_hi = jnp.concatenate([user_emb[:, HD:], tgt_item_emb[:, HD:]], axis=0)

    zc = jnp.zeros((ZROWS, HD), f32)
    zc16 = jnp.zeros((ROWS_PER_TILE, 16), f32)

    deg_s, deg_t = _deg_sc(d_src, d_tgt, zc16)

    ws0, ws1 = src_conv_W[0], src_conv_W[1]
    wt0, wt1 = tgt_conv_W[0], tgt_conv_W[1]
    bs0 = src_conv_b[0].reshape(1, D)
    bs1 = src_conv_b[1].reshape(1, D)
    bt0 = tgt_conv_b[0].reshape(1, D)
    bt1 = tgt_conv_b[1].reshape(1, D)
    mw0lo, mw0hi = mix_W[0][:D], mix_W[0][D:]
    mw1lo, mw1hi = mix_W[1][:D], mix_W[1][D:]
    mb0 = mix_b[0].reshape(1, D)
    mb1 = mix_b[1].reshape(1, D)

    as_lo, as_hi = _conv_sc(s_src, d_src, xs_lo, xs_hi, zc, deg_s)
    at_lo, at_hi = _conv_sc(s_tgt, d_tgt, xt_lo, xt_hi, zc, deg_s)
    x1s_lo, x1s_hi, x1t_lo, x1t_hi = _layer_tc(
        as_lo, as_hi, at_lo, at_hi, deg_s, deg_t,
        ws0, bs0, wt0, bt0, mw0lo, mw0hi, mb0)

    a2s_lo, a2s_hi = _conv_sc(s_src, d_src, x1s_lo, x1s_hi, zc, deg_s)
    a2t_lo, a2t_hi = _conv_sc(s_tgt, d_tgt, x1t_lo, x1t_hi, zc, deg_s)
    pu, pi = _mix_pred_tc(a2s_lo, a2s_hi, a2t_lo, a2t_hi, deg_s, deg_t,
                          ws1, bs1, wt1, bt1, mw1lo, mw1hi, mb1,
                          user_emb, x1s_lo, x1s_hi, pred_W, pred_b.reshape(1, 1))

    out = _head_sc(pu.reshape(N_USERS), pi.reshape(N_USERS), l0, l1)
    return out.reshape(B_LINKS, 1)
